# baseline (speedup 1.0000x reference)
import jax
import jax.numpy as jnp
from jax.experimental import pallas as pl

N = 10000
EMBED = 128
HIDDEN = 256
HEADS = 4
HC = HIDDEN * HEADS
NUM_CLASSES = 64
K = 8
TW = 1.0


def _final_mm_body(h_ref, w_ref, b_ref, o_ref):
    o_ref[...] = jnp.dot(h_ref[...], w_ref[...],
                         preferred_element_type=jnp.float32) + b_ref[...]


def _final_mm(h, Wf, bf):
    return pl.pallas_call(
        _final_mm_body,
        out_shape=jax.ShapeDtypeStruct((h.shape[0], NUM_CLASSES), jnp.float32),
    )(h, Wf, bf[None, :])


def _layer_norm(h, g, b, eps=1e-5):
    mu = h.mean(axis=-1, keepdims=True)
    var = ((h - mu) ** 2).mean(axis=-1, keepdims=True)
    return (h - mu) / jnp.sqrt(var + eps) * g + b


def _build_edges(xb):
    xn = xb / (jnp.linalg.norm(xb, axis=1, keepdims=True) + 1e-8)
    sim = xn @ xn.T
    n = xb.shape[0]
    _, idx = jax.lax.top_k(sim, K + 1)
    nbrs = idx[:, 1:]
    src = jnp.repeat(jnp.arange(n), K).astype(jnp.int32)
    dst = nbrs.reshape(-1).astype(jnp.int32)
    w = sim[src, dst] + TW * (jnp.abs(dst - src) == 1).astype(jnp.float32)
    has_next = (nbrs[:-1, :] == (jnp.arange(n - 1)[:, None] + 1)).any(axis=1)
    missing_mask = ~has_next
    base = jnp.arange(n - 1, dtype=jnp.int32)
    esrc = jnp.concatenate([base, base + 1])
    edst = jnp.concatenate([base + 1, base])
    emask = jnp.concatenate([missing_mask, missing_mask])
    mask = jnp.concatenate([jnp.ones(src.shape, jnp.bool_), emask])
    src = jnp.concatenate([src, esrc])
    dst = jnp.concatenate([dst, edst])
    w = jnp.concatenate([w, jnp.full((esrc.shape[0],), TW, jnp.float32)])
    return src, dst, w, mask


def _gat_conv(x, src, dst, ew, mask, W, a_s, a_d, b, concat):
    n = x.shape[0]
    h = (x @ W).reshape(n, HEADS, HIDDEN)
    al_s = (h * a_s[None, :, :]).sum(-1)
    al_d = (h * a_d[None, :, :]).sum(-1)
    e = al_s[src] + al_d[dst]
    e = jnp.where(e > 0, e, 0.2 * e)
    e = jnp.where(mask[:, None], e, -jnp.inf)
    m = jax.ops.segment_max(e, dst, num_segments=n)
    m = jax.lax.stop_gradient(jnp.where(jnp.isfinite(m), m, 0.0))
    ex = jnp.exp(e - m[dst])
    denom = jax.ops.segment_sum(ex, dst, num_segments=n)
    alpha = ex / (denom[dst] + 1e-16)
    msg = h[src] * alpha[:, :, None] * ew[:, None, None]
    out = jax.ops.segment_sum(msg, dst, num_segments=n)
    out = out.reshape(n, HC) if concat else out.mean(axis=1)
    return out + b


def kernel(x, bn_g, bn_b, bn_mean, bn_var, W1, as1, ad1, b1, W2, as2, ad2, b2,
           W3, as3, ad3, b3, W4, as4, ad4, b4, Wr1, br1, Wr2, br2, g1, be1,
           g2, be2, g3, be3, g4, be4, Wf, bf):
    xb = (x - bn_mean) / jnp.sqrt(bn_var + 1e-5) * bn_g + bn_b
    src, dst, ew, mask = _build_edges(xb)
    r = xb @ Wr1 + br1
    h = _gat_conv(xb, src, dst, ew, mask, W1, as1, ad1, b1, True)
    h = _layer_norm(jax.nn.relu(h) + r, g1, be1)
    r = h
    h = _gat_conv(h, src, dst, ew, mask, W2, as2, ad2, b2, True)
    h = _layer_norm(jax.nn.relu(h) + r, g2, be2)
    r = h @ Wr2 + br2
    h = _gat_conv(h, src, dst, ew, mask, W3, as3, ad3, b3, False)
    h = _layer_norm(jax.nn.relu(h) + r, g3, be3)
    r = h
    h = _gat_conv(h, src, dst, ew, mask, W4, as4, ad4, b4, False)
    h = _layer_norm(jax.nn.relu(h) + r, g4, be4)
    return _final_mm(h, Wf, bf)


# speedup vs baseline: 1.4039x; 1.4039x over previous
import functools

import jax
import jax.numpy as jnp
from jax import lax
from jax.experimental import pallas as pl
from jax.experimental.pallas import tpu as pltpu

N = 10000
EMBED = 128
HIDDEN = 256
HEADS = 4
HC = HIDDEN * HEADS
NUM_CLASSES = 64
K = 8
TW = 1.0

NP = 10240
RB = 256
CB = 512
EP = 102400
NEG = -1e30


def _prep_body(x_ref, g_ref, b_ref, m_ref, v_ref, xb_ref, xn_ref):
    x = x_ref[...]
    xb = (x - m_ref[...]) / jnp.sqrt(v_ref[...] + 1e-5) * g_ref[...] + b_ref[...]
    nrm = jnp.sqrt((xb * xb).sum(axis=1, keepdims=True))
    xb_ref[...] = xb
    xn_ref[...] = xb / (nrm + 1e-8)


def _prep(x, bn_g, bn_b, bn_mean, bn_var):
    grid = (NP // RB,)
    return pl.pallas_call(
        _prep_body,
        grid=grid,
        in_specs=[pl.BlockSpec((RB, EMBED), lambda i: (i, 0))] +
                 [pl.BlockSpec((1, EMBED), lambda i: (0, 0))] * 4,
        out_specs=[pl.BlockSpec((RB, EMBED), lambda i: (i, 0))] * 2,
        out_shape=[jax.ShapeDtypeStruct((NP, EMBED), jnp.float32)] * 2,
    )(x, bn_g[None], bn_b[None], bn_mean[None], bn_var[None])


def _simtopk_body(xr_ref, xc_ref, nbr_ref, ew_ref, hn_ref, cv_ref, ci_ref):
    i = pl.program_id(0)
    j = pl.program_id(1)

    @pl.when(j == 0)
    def _init():
        cv_ref[...] = jnp.full((RB, K), NEG, jnp.float32)
        ci_ref[...] = jnp.zeros((RB, K), jnp.int32)

    s = lax.dot_general(xr_ref[...], xc_ref[...], (((1,), (1,)), ((), ())),
                        preferred_element_type=jnp.float32)
    rowid = i * RB + lax.broadcasted_iota(jnp.int32, (RB, CB), 0)
    colid = j * CB + lax.broadcasted_iota(jnp.int32, (RB, CB), 1)
    s = jnp.where((colid == rowid) | (colid >= N), NEG, s)

    v = jnp.concatenate([cv_ref[...], s], axis=1)
    idxs = jnp.concatenate([ci_ref[...], colid], axis=1)
    lane = lax.broadcasted_iota(jnp.int32, (RB, K + CB), 1)
    nv, ni = [], []
    for _ in range(K):
        am = jnp.argmax(v, axis=1)[:, None]
        hit = lane == am
        nv.append(jnp.sum(jnp.where(hit, v, 0.0), axis=1))
        ni.append(jnp.sum(jnp.where(hit, idxs, 0), axis=1))
        v = jnp.where(hit, NEG, v)
    cv_ref[...] = jnp.stack(nv, axis=1)
    ci_ref[...] = jnp.stack(ni, axis=1)

    @pl.when(j == (NP // CB) - 1)
    def _fin():
        nb = ci_ref[...]
        vv = cv_ref[...]
        rid = i * RB + lax.broadcasted_iota(jnp.int32, (RB, K), 0)
        nbr_ref[...] = nb
        ew_ref[...] = vv + TW * (jnp.abs(nb - rid) == 1).astype(jnp.float32)
        hn = (nb == rid + 1).any(axis=1, keepdims=True)
        hn_ref[...] = jnp.broadcast_to(hn, (RB, K)).astype(jnp.int32)


def _simtopk(xn):
    grid = (NP // RB, NP // CB)
    return pl.pallas_call(
        _simtopk_body,
        grid=grid,
        in_specs=[pl.BlockSpec((RB, EMBED), lambda i, j: (i, 0)),
                  pl.BlockSpec((CB, EMBED), lambda i, j: (j, 0))],
        out_specs=[pl.BlockSpec((RB, K), lambda i, j: (i, 0))] * 3,
        out_shape=[jax.ShapeDtypeStruct((NP, K), jnp.int32),
                   jax.ShapeDtypeStruct((NP, K), jnp.float32),
                   jax.ShapeDtypeStruct((NP, K), jnp.int32)],
        scratch_shapes=[pltpu.VMEM((RB, K), jnp.float32),
                        pltpu.VMEM((RB, K), jnp.int32)],
    )(xn, xn)


def _mm_body(x_ref, w_ref, b_ref, o_ref):
    o_ref[0] = jnp.dot(x_ref[...], w_ref[...],
                       preferred_element_type=jnp.float32) + b_ref[0]


def _mm(x, Wcat, bias):
    Pn = Wcat.shape[1] // 128
    Kd = x.shape[1]
    grid = (NP // RB, Pn)
    return pl.pallas_call(
        _mm_body,
        grid=grid,
        in_specs=[pl.BlockSpec((RB, Kd), lambda i, j: (i, 0)),
                  pl.BlockSpec((Kd, 128), lambda i, j: (0, j)),
                  pl.BlockSpec((1, 1, 128), lambda i, j: (j, 0, 0))],
        out_specs=pl.BlockSpec((1, RB, 128), lambda i, j: (j, i, 0)),
        out_shape=jax.ShapeDtypeStruct((Pn, NP, 128), jnp.float32),
    )(x, Wcat, bias.reshape(Pn, 1, 128))


def _post12_body(agg_ref, r_ref, b_ref, g_ref, be_ref, o_ref, *, r_slabbed):
    vs = []
    ssum = 0.0
    ssq = 0.0
    for q in range(8):
        r = r_ref[q] if r_slabbed else r_ref[:, q * 128:(q + 1) * 128]
        v = jnp.maximum(agg_ref[q] + b_ref[q], 0.0) + r
        vs.append(v)
        ssum = ssum + v.sum(axis=1, keepdims=True)
        ssq = ssq + (v * v).sum(axis=1, keepdims=True)
    mu = ssum / HC
    var = ssq / HC - mu * mu
    rstd = lax.rsqrt(var + 1e-5)
    for q in range(8):
        o_ref[:, q * 128:(q + 1) * 128] = (vs[q] - mu) * rstd * g_ref[q] + be_ref[q]


def _post12(agg, r, r_spec, b, g, be, r_slabbed):
    grid = (NP // RB,)
    return pl.pallas_call(
        functools.partial(_post12_body, r_slabbed=r_slabbed),
        grid=grid,
        in_specs=[pl.BlockSpec((8, RB, 128), lambda i: (0, i, 0)),
                  r_spec,
                  pl.BlockSpec((8, 1, 128), lambda i: (0, 0, 0)),
                  pl.BlockSpec((8, 1, 128), lambda i: (0, 0, 0)),
                  pl.BlockSpec((8, 1, 128), lambda i: (0, 0, 0))],
        out_specs=pl.BlockSpec((RB, HC), lambda i: (i, 0)),
        out_shape=jax.ShapeDtypeStruct((NP, HC), jnp.float32),
    )(agg, r, b.reshape(8, 1, 128), g.reshape(8, 1, 128), be.reshape(8, 1, 128))


def _post34_body(agg_ref, r_ref, b_ref, g_ref, be_ref, wf_ref, bf_ref, o_ref,
                 *, r_slabbed, final):
    vs = []
    ssum = 0.0
    ssq = 0.0
    for p in range(2):
        m = 0.25 * (agg_ref[p] + agg_ref[2 + p] + agg_ref[4 + p] + agg_ref[6 + p])
        r = r_ref[p] if r_slabbed else r_ref[:, p * 128:(p + 1) * 128]
        v = jnp.maximum(m + b_ref[p], 0.0) + r
        vs.append(v)
        ssum = ssum + v.sum(axis=1, keepdims=True)
        ssq = ssq + (v * v).sum(axis=1, keepdims=True)
    mu = ssum / HIDDEN
    var = ssq / HIDDEN - mu * mu
    rstd = lax.rsqrt(var + 1e-5)
    hs = [(vs[p] - mu) * rstd * g_ref[p] + be_ref[p] for p in range(2)]
    if final:
        acc = jnp.dot(hs[0], wf_ref[0:128], preferred_element_type=jnp.float32)
        acc = acc + jnp.dot(hs[1], wf_ref[128:256],
                            preferred_element_type=jnp.float32)
        o_ref[...] = acc + bf_ref[...]
    else:
        o_ref[:, 0:128] = hs[0]
        o_ref[:, 128:256] = hs[1]


def _post34(agg, r, r_spec, b, g, be, r_slabbed, final, wf_pad, bf_pad):
    grid = (NP // RB,)
    ocols = 128 if final else HIDDEN
    return pl.pallas_call(
        functools.partial(_post34_body, r_slabbed=r_slabbed, final=final),
        grid=grid,
        in_specs=[pl.BlockSpec((8, RB, 128), lambda i: (0, i, 0)),
                  r_spec,
                  pl.BlockSpec((2, 1, 128), lambda i: (0, 0, 0)),
                  pl.BlockSpec((2, 1, 128), lambda i: (0, 0, 0)),
                  pl.BlockSpec((2, 1, 128), lambda i: (0, 0, 0)),
                  pl.BlockSpec((HIDDEN, 128), lambda i: (0, 0)),
                  pl.BlockSpec((1, 128), lambda i: (0, 0))],
        out_specs=pl.BlockSpec((RB, ocols), lambda i: (i, 0)),
        out_shape=jax.ShapeDtypeStruct((NP, ocols), jnp.float32),
    )(agg, r, b.reshape(2, 1, 128), g.reshape(2, 1, 128),
      be.reshape(2, 1, 128), wf_pad, bf_pad)


def _edge_phase(hsl, als, ald, esrc, edst, ew, emask):
    e = als[esrc] + ald[edst]
    e = jnp.where(e > 0, e, 0.2 * e)
    ex = jnp.where(emask[:, None], jnp.exp(e), 0.0)
    denom = jax.ops.segment_sum(ex, edst, num_segments=NP)
    coef = ex * ew[:, None] / (denom[edst] + 1e-16)
    h = jnp.moveaxis(hsl[0:8], 0, 1).reshape(NP, 8, 128)
    msg = h[esrc] * coef[:, [0, 0, 1, 1, 2, 2, 3, 3]][:, :, None]
    agg = jax.ops.segment_sum(msg, edst, num_segments=NP)
    return jnp.moveaxis(agg, 1, 0)


def _build_attn_mat(a_s, a_d):
    A = jnp.zeros((HC, 16), jnp.float32)
    hid = jnp.arange(HC) // HIDDEN
    pos = jnp.arange(HC) % HIDDEN
    As = a_s[hid, pos]
    Ad = a_d[hid, pos]
    onehot = (jnp.arange(16)[None, :] == hid[:, None]).astype(jnp.float32)
    onehot4 = (jnp.arange(16)[None, :] == (hid[:, None] + 4)).astype(jnp.float32)
    A = onehot * As[:, None] + onehot4 * Ad[:, None]
    return A


def _wcat(W, a_s, a_d, Wr=None):
    A = _build_attn_mat(a_s, a_d)
    Apad = jnp.pad(W @ A, ((0, 0), (0, 112)))
    parts = [W]
    if Wr is not None:
        parts.append(Wr)
    parts.append(Apad)
    return jnp.concatenate(parts, axis=1)


def kernel(x, bn_g, bn_b, bn_mean, bn_var, W1, as1, ad1, b1, W2, as2, ad2, b2,
           W3, as3, ad3, b3, W4, as4, ad4, b4, Wr1, br1, Wr2, br2, g1, be1,
           g2, be2, g3, be3, g4, be4, Wf, bf):
    xpad = jnp.pad(x, ((0, NP - N), (0, 0)))
    xb, xn = _prep(xpad, bn_g, bn_b, bn_mean, bn_var)
    nbr, ewk, hn = _simtopk(xn)
    nbr = nbr[:N]
    ewk = ewk[:N]
    missing = hn[:N - 1, 0] == 0

    base = jnp.arange(N - 1, dtype=jnp.int32)
    esrc = jnp.concatenate([
        jnp.repeat(jnp.arange(N, dtype=jnp.int32), K), base, base + 1,
        jnp.zeros((EP - N * K - 2 * (N - 1),), jnp.int32)])
    edst = jnp.concatenate([
        nbr.reshape(-1), base + 1, base,
        jnp.zeros((EP - N * K - 2 * (N - 1),), jnp.int32)])
    ew = jnp.concatenate([
        ewk.reshape(-1), jnp.full((2 * (N - 1),), TW, jnp.float32),
        jnp.zeros((EP - N * K - 2 * (N - 1),), jnp.float32)])
    emask = jnp.concatenate([
        jnp.ones((N * K,), jnp.bool_), missing, missing,
        jnp.zeros((EP - N * K - 2 * (N - 1),), jnp.bool_)])

    zero8 = jnp.zeros((8 * 128,), jnp.float32)
    zero128 = jnp.zeros((128,), jnp.float32)

    Wc = _wcat(W1, as1, ad1, Wr1)
    bias = jnp.concatenate([zero8, br1, zero128])
    hm = _mm(xb, Wc, bias)
    als, ald = hm[16, :, 0:4], hm[16, :, 4:8]
    agg = _edge_phase(hm, als, ald, esrc, edst, ew, emask)
    rspec = pl.BlockSpec((8, RB, 128), lambda i: (1, i, 0))
    h = _post12(agg, hm, rspec, b1, g1, be1, r_slabbed=True)

    Wc = _wcat(W2, as2, ad2)
    bias = jnp.concatenate([zero8, zero128])
    hm = _mm(h, Wc, bias)
    als, ald = hm[8, :, 0:4], hm[8, :, 4:8]
    agg = _edge_phase(hm, als, ald, esrc, edst, ew, emask)
    rspec = pl.BlockSpec((RB, HC), lambda i: (i, 0))
    h = _post12(agg, h, rspec, b2, g2, be2, r_slabbed=False)

    Wc = _wcat(W3, as3, ad3, Wr2)
    bias = jnp.concatenate([zero8, br2, zero128])
    hm = _mm(h, Wc, bias)
    als, ald = hm[10, :, 0:4], hm[10, :, 4:8]
    agg = _edge_phase(hm, als, ald, esrc, edst, ew, emask)
    rspec = pl.BlockSpec((2, RB, 128), lambda i: (4, i, 0))
    wf_pad = jnp.pad(Wf, ((0, 0), (0, 128 - NUM_CLASSES)))
    bf_pad = jnp.pad(bf, (0, 128 - NUM_CLASSES))[None]
    h = _post34(agg, hm, rspec, b3, g3, be3, True, False, wf_pad, bf_pad)

    Wc = _wcat(W4, as4, ad4)
    bias = jnp.concatenate([zero8, zero128])
    hm = _mm(h, Wc, bias)
    als, ald = hm[8, :, 0:4], hm[8, :, 4:8]
    agg = _edge_phase(hm, als, ald, esrc, edst, ew, emask)
    rspec = pl.BlockSpec((RB, HIDDEN), lambda i: (i, 0))
    out = _post34(agg, h, rspec, b4, g4, be4, False, True, wf_pad, bf_pad)

    return out[:N, :NUM_CLASSES]


# speedup vs baseline: 6.8408x; 4.8726x over previous
import functools

import jax
import jax.numpy as jnp
from jax import lax
from jax.experimental import pallas as pl
from jax.experimental.pallas import tpu as pltpu
from jax.experimental.pallas import tpu_sc as plsc

N = 10000
EMBED = 128
HIDDEN = 256
HEADS = 4
HC = HIDDEN * HEADS
NUM_CLASSES = 64
K = 8
TW = 1.0

NP = 10240
RB = 256
CB = 512
EP = 102400
NEG = -1e30


def _prep_body(x_ref, g_ref, b_ref, m_ref, v_ref, xb_ref, xn_ref):
    x = x_ref[...]
    xb = (x - m_ref[...]) / jnp.sqrt(v_ref[...] + 1e-5) * g_ref[...] + b_ref[...]
    nrm = jnp.sqrt((xb * xb).sum(axis=1, keepdims=True))
    xb_ref[...] = xb
    xn_ref[...] = xb / (nrm + 1e-8)


def _prep(x, bn_g, bn_b, bn_mean, bn_var):
    grid = (NP // RB,)
    return pl.pallas_call(
        _prep_body,
        grid=grid,
        in_specs=[pl.BlockSpec((RB, EMBED), lambda i: (i, 0))] +
                 [pl.BlockSpec((1, EMBED), lambda i: (0, 0))] * 4,
        out_specs=[pl.BlockSpec((RB, EMBED), lambda i: (i, 0))] * 2,
        out_shape=[jax.ShapeDtypeStruct((NP, EMBED), jnp.float32)] * 2,
    )(x, bn_g[None], bn_b[None], bn_mean[None], bn_var[None])


def _simtopk_body(xr_ref, xc_ref, nbr_ref, ew_ref, hn_ref, cv_ref, ci_ref):
    i = pl.program_id(0)
    j = pl.program_id(1)

    @pl.when(j == 0)
    def _init():
        cv_ref[...] = jnp.full((RB, K), NEG, jnp.float32)
        ci_ref[...] = jnp.zeros((RB, K), jnp.int32)

    s = lax.dot_general(xr_ref[...], xc_ref[...], (((1,), (1,)), ((), ())),
                        preferred_element_type=jnp.float32)
    rowid = i * RB + lax.broadcasted_iota(jnp.int32, (RB, CB), 0)
    colid = j * CB + lax.broadcasted_iota(jnp.int32, (RB, CB), 1)
    s = jnp.where((colid == rowid) | (colid >= N), NEG, s)

    v = jnp.concatenate([cv_ref[...], s], axis=1)
    idxs = jnp.concatenate([ci_ref[...], colid], axis=1)
    lane = lax.broadcasted_iota(jnp.int32, (RB, K + CB), 1)
    nv, ni = [], []
    for _ in range(K):
        am = jnp.argmax(v, axis=1)[:, None]
        hit = lane == am
        nv.append(jnp.sum(jnp.where(hit, v, 0.0), axis=1))
        ni.append(jnp.sum(jnp.where(hit, idxs, 0), axis=1))
        v = jnp.where(hit, NEG, v)
    cv_ref[...] = jnp.stack(nv, axis=1)
    ci_ref[...] = jnp.stack(ni, axis=1)

    @pl.when(j == (NP // CB) - 1)
    def _fin():
        nb = ci_ref[...]
        vv = cv_ref[...]
        rid = i * RB + lax.broadcasted_iota(jnp.int32, (RB, K), 0)
        nbr_ref[...] = nb
        ew_ref[...] = vv + TW * (jnp.abs(nb - rid) == 1).astype(jnp.float32)
        hn = (nb == rid + 1).any(axis=1, keepdims=True)
        hn_ref[...] = jnp.broadcast_to(hn, (RB, K)).astype(jnp.int32)


def _simtopk(xn):
    grid = (NP // RB, NP // CB)
    return pl.pallas_call(
        _simtopk_body,
        grid=grid,
        in_specs=[pl.BlockSpec((RB, EMBED), lambda i, j: (i, 0)),
                  pl.BlockSpec((CB, EMBED), lambda i, j: (j, 0))],
        out_specs=[pl.BlockSpec((RB, K), lambda i, j: (i, 0))] * 3,
        out_shape=[jax.ShapeDtypeStruct((NP, K), jnp.int32),
                   jax.ShapeDtypeStruct((NP, K), jnp.float32),
                   jax.ShapeDtypeStruct((NP, K), jnp.int32)],
        scratch_shapes=[pltpu.VMEM((RB, K), jnp.float32),
                        pltpu.VMEM((RB, K), jnp.int32)],
    )(xn, xn)


def _mm_body(x_ref, w_ref, b_ref, o_ref):
    o_ref[0] = jnp.dot(x_ref[...], w_ref[...],
                       preferred_element_type=jnp.float32) + b_ref[0]


def _mm(x, Wcat, bias):
    Pn = Wcat.shape[1] // 128
    Kd = x.shape[1]
    grid = (NP // RB, Pn)
    return pl.pallas_call(
        _mm_body,
        grid=grid,
        in_specs=[pl.BlockSpec((RB, Kd), lambda i, j: (i, 0)),
                  pl.BlockSpec((Kd, 128), lambda i, j: (0, j)),
                  pl.BlockSpec((1, 1, 128), lambda i, j: (j, 0, 0))],
        out_specs=pl.BlockSpec((1, RB, 128), lambda i, j: (j, i, 0)),
        out_shape=jax.ShapeDtypeStruct((Pn, NP, 128), jnp.float32),
    )(x, Wcat, bias.reshape(Pn, 1, 128))


def _post12_body(agg_ref, r_ref, b_ref, g_ref, be_ref, o_ref, *, r_slabbed):
    vs = []
    ssum = 0.0
    ssq = 0.0
    for q in range(8):
        r = r_ref[q] if r_slabbed else r_ref[:, q * 128:(q + 1) * 128]
        v = jnp.maximum(agg_ref[q] + b_ref[q], 0.0) + r
        vs.append(v)
        ssum = ssum + v.sum(axis=1, keepdims=True)
        ssq = ssq + (v * v).sum(axis=1, keepdims=True)
    mu = ssum / HC
    var = ssq / HC - mu * mu
    rstd = lax.rsqrt(var + 1e-5)
    for q in range(8):
        o_ref[:, q * 128:(q + 1) * 128] = (vs[q] - mu) * rstd * g_ref[q] + be_ref[q]


def _post12(agg, r, r_spec, b, g, be, r_slabbed):
    grid = (NP // RB,)
    return pl.pallas_call(
        functools.partial(_post12_body, r_slabbed=r_slabbed),
        grid=grid,
        in_specs=[pl.BlockSpec((8, RB, 128), lambda i: (0, i, 0)),
                  r_spec,
                  pl.BlockSpec((8, 1, 128), lambda i: (0, 0, 0)),
                  pl.BlockSpec((8, 1, 128), lambda i: (0, 0, 0)),
                  pl.BlockSpec((8, 1, 128), lambda i: (0, 0, 0))],
        out_specs=pl.BlockSpec((RB, HC), lambda i: (i, 0)),
        out_shape=jax.ShapeDtypeStruct((NP, HC), jnp.float32),
    )(agg, r, b.reshape(8, 1, 128), g.reshape(8, 1, 128), be.reshape(8, 1, 128))


def _post34_body(agg_ref, r_ref, b_ref, g_ref, be_ref, wf_ref, bf_ref, o_ref,
                 *, r_slabbed, final):
    vs = []
    ssum = 0.0
    ssq = 0.0
    for p in range(2):
        m = 0.25 * (agg_ref[p] + agg_ref[2 + p] + agg_ref[4 + p] + agg_ref[6 + p])
        r = r_ref[p] if r_slabbed else r_ref[:, p * 128:(p + 1) * 128]
        v = jnp.maximum(m + b_ref[p], 0.0) + r
        vs.append(v)
        ssum = ssum + v.sum(axis=1, keepdims=True)
        ssq = ssq + (v * v).sum(axis=1, keepdims=True)
    mu = ssum / HIDDEN
    var = ssq / HIDDEN - mu * mu
    rstd = lax.rsqrt(var + 1e-5)
    hs = [(vs[p] - mu) * rstd * g_ref[p] + be_ref[p] for p in range(2)]
    if final:
        acc = jnp.dot(hs[0], wf_ref[0:128], preferred_element_type=jnp.float32)
        acc = acc + jnp.dot(hs[1], wf_ref[128:256],
                            preferred_element_type=jnp.float32)
        o_ref[...] = acc + bf_ref[...]
    else:
        o_ref[:, 0:128] = hs[0]
        o_ref[:, 128:256] = hs[1]


def _post34(agg, r, r_spec, b, g, be, r_slabbed, final, wf_pad, bf_pad):
    grid = (NP // RB,)
    ocols = 128 if final else HIDDEN
    return pl.pallas_call(
        functools.partial(_post34_body, r_slabbed=r_slabbed, final=final),
        grid=grid,
        in_specs=[pl.BlockSpec((8, RB, 128), lambda i: (0, i, 0)),
                  r_spec,
                  pl.BlockSpec((2, 1, 128), lambda i: (0, 0, 0)),
                  pl.BlockSpec((2, 1, 128), lambda i: (0, 0, 0)),
                  pl.BlockSpec((2, 1, 128), lambda i: (0, 0, 0)),
                  pl.BlockSpec((HIDDEN, 128), lambda i: (0, 0)),
                  pl.BlockSpec((1, 128), lambda i: (0, 0))],
        out_specs=pl.BlockSpec((RB, ocols), lambda i: (i, 0)),
        out_shape=jax.ShapeDtypeStruct((NP, ocols), jnp.float32),
    )(agg, r, b.reshape(2, 1, 128), g.reshape(2, 1, 128),
      be.reshape(2, 1, 128), wf_pad, bf_pad)


NT = 16
ECH = 128
ROWS_PT = None


def _sc_mesh():
    return plsc.VectorSubcoreMesh(core_axis_name="c", subcore_axis_name="s")


def _sc_params():
    return pltpu.CompilerParams(needs_layout_passes=False)


def _i16():
    return lax.iota(jnp.int32, 16)


def _p1_body(tabs, esrc, edst, ew, emaskf, coefT,
             t0, t1, t2, t3, den0, den1, exb0, exb1, sidx_v, didx_v, mk_v,
             ewv_v, cf0_v, cf1_v, rbuf_v, acc_v, part0, part1, fin0, fin1):
    ept = EP // NT
    nch = ept // ECH
    rpt = NP // NT
    c = lax.axis_index("c")
    s = lax.axis_index("s")
    lane = _i16()

    pltpu.sync_copy(tabs.at[c].at[0], t0)
    pltpu.sync_copy(tabs.at[c].at[1], t1)
    pltpu.sync_copy(tabs.at[c].at[2], t2)
    pltpu.sync_copy(tabs.at[c].at[3], t3)

    def _zd(i, _):
        den0[pl.ds(i * 16, 16)] = jnp.zeros((16,), jnp.float32)
        den1[pl.ds(i * 16, 16)] = jnp.zeros((16,), jnp.float32)
        return 0
    lax.fori_loop(0, NP // 16, _zd, 0)

    def _chunk_b(ch, _):
        off = s * ept + ch * ECH
        pltpu.sync_copy(esrc.at[pl.ds(off, ECH)], sidx_v)
        pltpu.sync_copy(edst.at[pl.ds(off, ECH)], didx_v)
        pltpu.sync_copy(emaskf.at[pl.ds(off, ECH)], mk_v)
        for g in range(8):
            si = sidx_v[pl.ds(g * 16, 16)]
            di = didx_v[pl.ds(g * 16, 16)]
            mk = mk_v[pl.ds(g * 16, 16)]
            for hl, (ta, td, exb, den) in enumerate(((t0, t2, exb0, den0),
                                                     (t1, t3, exb1, den1))):
                e = plsc.load_gather(ta, [si]) + plsc.load_gather(td, [di])
                e = jnp.where(e > 0.0, e, 0.2 * e)
                ex = jnp.exp(e) * mk
                exb[pl.ds(ch * ECH + g * 16, 16)] = ex
                for l in range(16):
                    plsc.addupdate_scatter(den, [di], ex, mask=lane == l)
        return 0
    lax.fori_loop(0, nch, _chunk_b, 0)

    pltpu.sync_copy(den0, part0.at[s])
    pltpu.sync_copy(den1, part1.at[s])
    plsc.subcore_barrier()
    for den, part, fin in ((den0, part0, fin0), (den1, part1, fin1)):
        def _za(i, _):
            acc_v[pl.ds(i * 16, 16)] = jnp.zeros((16,), jnp.float32)
            return 0
        lax.fori_loop(0, rpt // 16, _za, 0)
        for p in range(NT):
            pltpu.sync_copy(part.at[p].at[pl.ds(s * rpt, rpt)], rbuf_v)

            def _acc(g, _):
                acc_v[pl.ds(g * 16, 16)] = (acc_v[pl.ds(g * 16, 16)]
                                            + rbuf_v[pl.ds(g * 16, 16)])
                return 0
            lax.fori_loop(0, rpt // 16, _acc, 0)
        pltpu.sync_copy(acc_v, fin.at[pl.ds(s * rpt, rpt)])
    plsc.subcore_barrier()
    pltpu.sync_copy(fin0, den0)
    pltpu.sync_copy(fin1, den1)

    def _chunk_c(ch, _):
        off = s * ept + ch * ECH
        pltpu.sync_copy(edst.at[pl.ds(off, ECH)], didx_v)
        pltpu.sync_copy(ew.at[pl.ds(off, ECH)], ewv_v)
        for g in range(8):
            ewg = ewv_v[pl.ds(g * 16, 16)]
            di = didx_v[pl.ds(g * 16, 16)]
            for hl, (exb, den, cfb) in enumerate(((exb0, den0, cf0_v),
                                                  (exb1, den1, cf1_v))):
                dv = plsc.load_gather(den, [di])
                ex = exb[pl.ds(ch * ECH + g * 16, 16)]
                cfb[pl.ds(g * 16, 16)] = ex * ewg / (dv + 1e-16)
        hg = 2 * c
        pltpu.sync_copy(cf0_v, coefT.at[pl.ds(hg * EP + off, ECH)])
        pltpu.sync_copy(cf1_v, coefT.at[pl.ds((hg + 1) * EP + off, ECH)])
        return 0
    lax.fori_loop(0, nch, _chunk_c, 0)


def _p1(tabs, esrc, edst, ew, emaskf):
    ept = EP // NT
    rpt = NP // NT
    f32 = jnp.float32
    i32 = jnp.int32
    fn = pl.kernel(
        _p1_body,
        out_type=jax.ShapeDtypeStruct((4 * EP,), f32),
        mesh=_sc_mesh(),
        compiler_params=_sc_params(),
        scratch_types=[
            pltpu.VMEM((NP,), f32), pltpu.VMEM((NP,), f32),
            pltpu.VMEM((NP,), f32), pltpu.VMEM((NP,), f32),
            pltpu.VMEM((NP,), f32), pltpu.VMEM((NP,), f32),
            pltpu.VMEM((ept,), f32), pltpu.VMEM((ept,), f32),
            pltpu.VMEM((ECH,), i32), pltpu.VMEM((ECH,), i32),
            pltpu.VMEM((ECH,), f32), pltpu.VMEM((ECH,), f32),
            pltpu.VMEM((ECH,), f32), pltpu.VMEM((ECH,), f32),
            pltpu.VMEM((rpt,), f32), pltpu.VMEM((rpt,), f32),
            pltpu.VMEM_SHARED((NT, NP), f32),
            pltpu.VMEM_SHARED((NT, NP), f32),
            pltpu.VMEM_SHARED((NP,), f32),
            pltpu.VMEM_SHARED((NP,), f32),
        ],
    )
    return fn(tabs, esrc, edst, ew, emaskf)


def _p3_body(hmflat, esrc, edst, coefT, aggflat,
             zb_v, rows_v, sidx_v, didx_v, cb_v, outS, sem):
    ept = EP // NT
    nch = ept // ECH
    rpt = NP // NT
    c = lax.axis_index("c")
    s = lax.axis_index("s")

    def _zb(i, _):
        for q in range(8):
            zb_v[i, pl.ds(q * 16, 16)] = jnp.zeros((16,), jnp.float32)
        return 0
    lax.fori_loop(0, ECH, _zb, 0)

    for s_local in range(4):
        slab = 4 * c + s_local
        head = 2 * c + (s_local // 2)
        for j in range(rpt // ECH):
            pltpu.sync_copy(zb_v, outS.at[pl.ds(s * rpt + j * ECH, ECH)])
        plsc.subcore_barrier()

        def _chunk(ch, _):
            off = s * ept + ch * ECH
            pltpu.sync_copy(esrc.at[pl.ds(off, ECH)], sidx_v)
            for g in range(8):
                sidx_v[pl.ds(g * 16, 16)] = (sidx_v[pl.ds(g * 16, 16)]
                                             + slab * NP)
            pltpu.sync_copy(edst.at[pl.ds(off, ECH)], didx_v)
            pltpu.sync_copy(coefT.at[pl.ds(head * EP + off, ECH)], cb_v)
            pltpu.async_copy(hmflat.at[sidx_v], rows_v, sem).wait()

            def _scale(r, _):
                spl = plsc.load_gather(cb_v, [jnp.full((16,), r, jnp.int32)])
                for q in range(8):
                    rows_v[r, pl.ds(q * 16, 16)] = (
                        rows_v[r, pl.ds(q * 16, 16)] * spl)
                return 0
            lax.fori_loop(0, ECH, _scale, 0)
            pltpu.sync_copy(rows_v, outS.at[didx_v], add=True)
            return 0
        lax.fori_loop(0, nch, _chunk, 0)
        plsc.subcore_barrier()

        for j in range(rpt // ECH):
            pltpu.sync_copy(
                outS.at[pl.ds(s * rpt + j * ECH, ECH)],
                aggflat.at[pl.ds(slab * NP + s * rpt + j * ECH, ECH)])
        plsc.subcore_barrier()


def _p3(hmflat, esrc, edst, coefT):
    f32 = jnp.float32
    i32 = jnp.int32
    fn = pl.kernel(
        _p3_body,
        out_type=jax.ShapeDtypeStruct((8 * NP, 128), f32),
        mesh=_sc_mesh(),
        compiler_params=_sc_params(),
        scratch_types=[
            pltpu.VMEM((ECH, 128), f32), pltpu.VMEM((ECH, 128), f32),
            pltpu.VMEM((ECH,), i32), pltpu.VMEM((ECH,), i32),
            pltpu.VMEM((ECH,), f32),
            pltpu.VMEM_SHARED((NP, 128), f32),
            pltpu.SemaphoreType.DMA,
        ],
    )
    return fn(hmflat, esrc, edst, coefT)


def _edge_phase(hm, esrc, edst, ew, emaskf, attn_panel):
    A8t = hm[attn_panel, :, 0:8].T
    tabs = jnp.stack([jnp.stack([A8t[0], A8t[1], A8t[4], A8t[5]]),
                      jnp.stack([A8t[2], A8t[3], A8t[6], A8t[7]])])
    coefT = _p1(tabs, esrc, edst, ew, emaskf)
    hmflat = hm.reshape(-1, 128)
    aggflat = _p3(hmflat, esrc, edst, coefT)
    return aggflat.reshape(8, NP, 128)


def _build_attn_mat(a_s, a_d):
    A = jnp.zeros((HC, 16), jnp.float32)
    hid = jnp.arange(HC) // HIDDEN
    pos = jnp.arange(HC) % HIDDEN
    As = a_s[hid, pos]
    Ad = a_d[hid, pos]
    onehot = (jnp.arange(16)[None, :] == hid[:, None]).astype(jnp.float32)
    onehot4 = (jnp.arange(16)[None, :] == (hid[:, None] + 4)).astype(jnp.float32)
    A = onehot * As[:, None] + onehot4 * Ad[:, None]
    return A


def _wcat(W, a_s, a_d, Wr=None):
    A = _build_attn_mat(a_s, a_d)
    Apad = jnp.pad(W @ A, ((0, 0), (0, 112)))
    parts = [W]
    if Wr is not None:
        parts.append(Wr)
    parts.append(Apad)
    return jnp.concatenate(parts, axis=1)


def kernel(x, bn_g, bn_b, bn_mean, bn_var, W1, as1, ad1, b1, W2, as2, ad2, b2,
           W3, as3, ad3, b3, W4, as4, ad4, b4, Wr1, br1, Wr2, br2, g1, be1,
           g2, be2, g3, be3, g4, be4, Wf, bf):
    xpad = jnp.pad(x, ((0, NP - N), (0, 0)))
    xb, xn = _prep(xpad, bn_g, bn_b, bn_mean, bn_var)
    nbr, ewk, hn = _simtopk(xn)
    nbr = nbr[:N]
    ewk = ewk[:N]
    missing = hn[:N - 1, 0] == 0

    base = jnp.arange(N - 1, dtype=jnp.int32)
    esrc = jnp.concatenate([
        jnp.repeat(jnp.arange(N, dtype=jnp.int32), K), base, base + 1,
        jnp.zeros((EP - N * K - 2 * (N - 1),), jnp.int32)])
    edst = jnp.concatenate([
        nbr.reshape(-1), base + 1, base,
        jnp.zeros((EP - N * K - 2 * (N - 1),), jnp.int32)])
    ew = jnp.concatenate([
        ewk.reshape(-1), jnp.full((2 * (N - 1),), TW, jnp.float32),
        jnp.zeros((EP - N * K - 2 * (N - 1),), jnp.float32)])
    missf = missing.astype(jnp.float32)
    emaskf = jnp.concatenate([
        jnp.ones((N * K,), jnp.float32), missf, missf,
        jnp.zeros((EP - N * K - 2 * (N - 1),), jnp.float32)])

    zero8 = jnp.zeros((8 * 128,), jnp.float32)
    zero128 = jnp.zeros((128,), jnp.float32)

    Wc = _wcat(W1, as1, ad1, Wr1)
    bias = jnp.concatenate([zero8, br1, zero128])
    hm = _mm(xb, Wc, bias)
    agg = _edge_phase(hm, esrc, edst, ew, emaskf, 16)
    rspec = pl.BlockSpec((8, RB, 128), lambda i: (1, i, 0))
    h = _post12(agg, hm, rspec, b1, g1, be1, r_slabbed=True)

    Wc = _wcat(W2, as2, ad2)
    bias = jnp.concatenate([zero8, zero128])
    hm = _mm(h, Wc, bias)
    agg = _edge_phase(hm, esrc, edst, ew, emaskf, 8)
    rspec = pl.BlockSpec((RB, HC), lambda i: (i, 0))
    h = _post12(agg, h, rspec, b2, g2, be2, r_slabbed=False)

    Wc = _wcat(W3, as3, ad3, Wr2)
    bias = jnp.concatenate([zero8, br2, zero128])
    hm = _mm(h, Wc, bias)
    agg = _edge_phase(hm, esrc, edst, ew, emaskf, 10)
    rspec = pl.BlockSpec((2, RB, 128), lambda i: (4, i, 0))
    wf_pad = jnp.pad(Wf, ((0, 0), (0, 128 - NUM_CLASSES)))
    bf_pad = jnp.pad(bf, (0, 128 - NUM_CLASSES))[None]
    h = _post34(agg, hm, rspec, b3, g3, be3, True, False, wf_pad, bf_pad)

    Wc = _wcat(W4, as4, ad4)
    bias = jnp.concatenate([zero8, zero128])
    hm = _mm(h, Wc, bias)
    agg = _edge_phase(hm, esrc, edst, ew, emaskf, 8)
    rspec = pl.BlockSpec((RB, HIDDEN), lambda i: (i, 0))
    out = _post34(agg, h, rspec, b4, g4, be4, False, True, wf_pad, bf_pad)

    return out[:N, :NUM_CLASSES]


# speedup vs baseline: 7.6061x; 1.1119x over previous
import functools

import jax
import jax.numpy as jnp
from jax import lax
from jax.experimental import pallas as pl
from jax.experimental.pallas import tpu as pltpu
from jax.experimental.pallas import tpu_sc as plsc

N = 10000
EMBED = 128
HIDDEN = 256
HEADS = 4
HC = HIDDEN * HEADS
NUM_CLASSES = 64
K = 8
TW = 1.0

NP = 10240
RB = 256
CB = 512
EP = 102400
NEG = -1e30


def _prep_body(x_ref, g_ref, b_ref, m_ref, v_ref, xb_ref, xn_ref):
    x = x_ref[...]
    xb = (x - m_ref[...]) / jnp.sqrt(v_ref[...] + 1e-5) * g_ref[...] + b_ref[...]
    nrm = jnp.sqrt((xb * xb).sum(axis=1, keepdims=True))
    xb_ref[...] = xb
    xn_ref[...] = xb / (nrm + 1e-8)


def _prep(x, bn_g, bn_b, bn_mean, bn_var):
    grid = (NP // RB,)
    return pl.pallas_call(
        _prep_body,
        grid=grid,
        in_specs=[pl.BlockSpec((RB, EMBED), lambda i: (i, 0))] +
                 [pl.BlockSpec((1, EMBED), lambda i: (0, 0))] * 4,
        out_specs=[pl.BlockSpec((RB, EMBED), lambda i: (i, 0))] * 2,
        out_shape=[jax.ShapeDtypeStruct((NP, EMBED), jnp.float32)] * 2,
    )(x, bn_g[None], bn_b[None], bn_mean[None], bn_var[None])


def _simtopk_body(xr_ref, xc_ref, nbr_ref, ew_ref, hn_ref, cv_ref, ci_ref):
    i = pl.program_id(0)
    j = pl.program_id(1)

    @pl.when(j == 0)
    def _init():
        cv_ref[...] = jnp.full((RB, K), NEG, jnp.float32)
        ci_ref[...] = jnp.zeros((RB, K), jnp.int32)

    s = lax.dot_general(xr_ref[...], xc_ref[...], (((1,), (1,)), ((), ())),
                        preferred_element_type=jnp.float32)
    rowid = i * RB + lax.broadcasted_iota(jnp.int32, (RB, CB), 0)
    colid = j * CB + lax.broadcasted_iota(jnp.int32, (RB, CB), 1)
    s = jnp.where((colid == rowid) | (colid >= N), NEG, s)

    v = jnp.concatenate([cv_ref[...], s], axis=1)
    idxs = jnp.concatenate([ci_ref[...], colid], axis=1)
    lane = lax.broadcasted_iota(jnp.int32, (RB, K + CB), 1)
    nv, ni = [], []
    for _ in range(K):
        am = jnp.argmax(v, axis=1)[:, None]
        hit = lane == am
        nv.append(jnp.sum(jnp.where(hit, v, 0.0), axis=1))
        ni.append(jnp.sum(jnp.where(hit, idxs, 0), axis=1))
        v = jnp.where(hit, NEG, v)
    cv_ref[...] = jnp.stack(nv, axis=1)
    ci_ref[...] = jnp.stack(ni, axis=1)

    @pl.when(j == (NP // CB) - 1)
    def _fin():
        nb = ci_ref[...]
        vv = cv_ref[...]
        rid = i * RB + lax.broadcasted_iota(jnp.int32, (RB, K), 0)
        nbr_ref[...] = nb
        ew_ref[...] = vv + TW * (jnp.abs(nb - rid) == 1).astype(jnp.float32)
        hn = (nb == rid + 1).any(axis=1, keepdims=True)
        hn_ref[...] = jnp.broadcast_to(hn, (RB, K)).astype(jnp.int32)


def _simtopk(xn):
    grid = (NP // RB, NP // CB)
    return pl.pallas_call(
        _simtopk_body,
        grid=grid,
        in_specs=[pl.BlockSpec((RB, EMBED), lambda i, j: (i, 0)),
                  pl.BlockSpec((CB, EMBED), lambda i, j: (j, 0))],
        out_specs=[pl.BlockSpec((RB, K), lambda i, j: (i, 0))] * 3,
        out_shape=[jax.ShapeDtypeStruct((NP, K), jnp.int32),
                   jax.ShapeDtypeStruct((NP, K), jnp.float32),
                   jax.ShapeDtypeStruct((NP, K), jnp.int32)],
        scratch_shapes=[pltpu.VMEM((RB, K), jnp.float32),
                        pltpu.VMEM((RB, K), jnp.int32)],
    )(xn, xn)


def _mm_body(x_ref, w_ref, b_ref, o_ref):
    o_ref[0] = jnp.dot(x_ref[...], w_ref[...],
                       preferred_element_type=jnp.float32) + b_ref[0]


def _mm(x, Wcat, bias):
    Pn = Wcat.shape[1] // 128
    Kd = x.shape[1]
    grid = (NP // RB, Pn)
    return pl.pallas_call(
        _mm_body,
        grid=grid,
        in_specs=[pl.BlockSpec((RB, Kd), lambda i, j: (i, 0)),
                  pl.BlockSpec((Kd, 128), lambda i, j: (0, j)),
                  pl.BlockSpec((1, 1, 128), lambda i, j: (j, 0, 0))],
        out_specs=pl.BlockSpec((1, RB, 128), lambda i, j: (j, i, 0)),
        out_shape=jax.ShapeDtypeStruct((Pn, NP, 128), jnp.float32),
    )(x, Wcat, bias.reshape(Pn, 1, 128))


def _post12_body(agg_ref, r_ref, b_ref, g_ref, be_ref, o_ref, *, r_slabbed):
    vs = []
    ssum = 0.0
    ssq = 0.0
    for q in range(8):
        r = r_ref[q] if r_slabbed else r_ref[:, q * 128:(q + 1) * 128]
        v = jnp.maximum(agg_ref[q] + b_ref[q], 0.0) + r
        vs.append(v)
        ssum = ssum + v.sum(axis=1, keepdims=True)
        ssq = ssq + (v * v).sum(axis=1, keepdims=True)
    mu = ssum / HC
    var = ssq / HC - mu * mu
    rstd = lax.rsqrt(var + 1e-5)
    for q in range(8):
        o_ref[:, q * 128:(q + 1) * 128] = (vs[q] - mu) * rstd * g_ref[q] + be_ref[q]


def _post12(agg, r, r_spec, b, g, be, r_slabbed):
    grid = (NP // RB,)
    return pl.pallas_call(
        functools.partial(_post12_body, r_slabbed=r_slabbed),
        grid=grid,
        in_specs=[pl.BlockSpec((8, RB, 128), lambda i: (0, i, 0)),
                  r_spec,
                  pl.BlockSpec((8, 1, 128), lambda i: (0, 0, 0)),
                  pl.BlockSpec((8, 1, 128), lambda i: (0, 0, 0)),
                  pl.BlockSpec((8, 1, 128), lambda i: (0, 0, 0))],
        out_specs=pl.BlockSpec((RB, HC), lambda i: (i, 0)),
        out_shape=jax.ShapeDtypeStruct((NP, HC), jnp.float32),
    )(agg, r, b.reshape(8, 1, 128), g.reshape(8, 1, 128), be.reshape(8, 1, 128))


def _post34_body(agg_ref, r_ref, b_ref, g_ref, be_ref, wf_ref, bf_ref, o_ref,
                 *, r_slabbed, final):
    vs = []
    ssum = 0.0
    ssq = 0.0
    for p in range(2):
        m = 0.25 * (agg_ref[p] + agg_ref[2 + p] + agg_ref[4 + p] + agg_ref[6 + p])
        r = r_ref[p] if r_slabbed else r_ref[:, p * 128:(p + 1) * 128]
        v = jnp.maximum(m + b_ref[p], 0.0) + r
        vs.append(v)
        ssum = ssum + v.sum(axis=1, keepdims=True)
        ssq = ssq + (v * v).sum(axis=1, keepdims=True)
    mu = ssum / HIDDEN
    var = ssq / HIDDEN - mu * mu
    rstd = lax.rsqrt(var + 1e-5)
    hs = [(vs[p] - mu) * rstd * g_ref[p] + be_ref[p] for p in range(2)]
    if final:
        acc = jnp.dot(hs[0], wf_ref[0:128], preferred_element_type=jnp.float32)
        acc = acc + jnp.dot(hs[1], wf_ref[128:256],
                            preferred_element_type=jnp.float32)
        o_ref[...] = acc + bf_ref[...]
    else:
        o_ref[:, 0:128] = hs[0]
        o_ref[:, 128:256] = hs[1]


def _post34(agg, r, r_spec, b, g, be, r_slabbed, final, wf_pad, bf_pad):
    grid = (NP // RB,)
    ocols = 128 if final else HIDDEN
    return pl.pallas_call(
        functools.partial(_post34_body, r_slabbed=r_slabbed, final=final),
        grid=grid,
        in_specs=[pl.BlockSpec((8, RB, 128), lambda i: (0, i, 0)),
                  r_spec,
                  pl.BlockSpec((2, 1, 128), lambda i: (0, 0, 0)),
                  pl.BlockSpec((2, 1, 128), lambda i: (0, 0, 0)),
                  pl.BlockSpec((2, 1, 128), lambda i: (0, 0, 0)),
                  pl.BlockSpec((HIDDEN, 128), lambda i: (0, 0)),
                  pl.BlockSpec((1, 128), lambda i: (0, 0))],
        out_specs=pl.BlockSpec((RB, ocols), lambda i: (i, 0)),
        out_shape=jax.ShapeDtypeStruct((NP, ocols), jnp.float32),
    )(agg, r, b.reshape(2, 1, 128), g.reshape(2, 1, 128),
      be.reshape(2, 1, 128), wf_pad, bf_pad)


NT = 16
ECH = 128
ROWS_PT = None


def _sc_mesh():
    return plsc.VectorSubcoreMesh(core_axis_name="c", subcore_axis_name="s")


def _sc_params():
    return pltpu.CompilerParams(needs_layout_passes=False)


def _i16():
    return lax.iota(jnp.int32, 16)


def _p1_body(tabs, esrc, edst, ew, emaskf, coefT, partH, finH,
             t0, t1, t2, t3, den0, den1, exb0, exb1, sidx_v, didx_v, mk_v,
             ewv_v, cf0_v, cf1_v, rbuf_v, acc_v):
    ept = EP // NT
    nch = ept // ECH
    rpt = NP // NT
    c = lax.axis_index("c")
    s = lax.axis_index("s")
    lane = _i16()

    pltpu.sync_copy(tabs.at[c].at[0], t0)
    pltpu.sync_copy(tabs.at[c].at[1], t1)
    pltpu.sync_copy(tabs.at[c].at[2], t2)
    pltpu.sync_copy(tabs.at[c].at[3], t3)

    def _zd(i, _):
        den0[pl.ds(i * 16, 16)] = jnp.zeros((16,), jnp.float32)
        den1[pl.ds(i * 16, 16)] = jnp.zeros((16,), jnp.float32)
        return 0
    lax.fori_loop(0, NP // 16, _zd, 0)

    def _chunk_b(ch, _):
        off = s * ept + ch * ECH
        pltpu.sync_copy(esrc.at[pl.ds(off, ECH)], sidx_v)
        pltpu.sync_copy(edst.at[pl.ds(off, ECH)], didx_v)
        pltpu.sync_copy(emaskf.at[pl.ds(off, ECH)], mk_v)
        for g in range(8):
            si = sidx_v[pl.ds(g * 16, 16)]
            di = didx_v[pl.ds(g * 16, 16)]
            mk = mk_v[pl.ds(g * 16, 16)]
            for hl, (ta, td, exb, den) in enumerate(((t0, t2, exb0, den0),
                                                     (t1, t3, exb1, den1))):
                e = plsc.load_gather(ta, [si]) + plsc.load_gather(td, [di])
                e = jnp.where(e > 0.0, e, 0.2 * e)
                ex = jnp.exp(e) * mk
                exb[pl.ds(ch * ECH + g * 16, 16)] = ex
                for l in range(16):
                    plsc.addupdate_scatter(den, [di], ex, mask=lane == l)
        return 0
    lax.fori_loop(0, nch, _chunk_b, 0)

    pltpu.sync_copy(den0, partH.at[pl.ds(((c * 2 + 0) * NT + s) * NP, NP)])
    pltpu.sync_copy(den1, partH.at[pl.ds(((c * 2 + 1) * NT + s) * NP, NP)])
    plsc.subcore_barrier()
    for hl, den in ((0, den0), (1, den1)):
        def _za(i, _):
            acc_v[pl.ds(i * 16, 16)] = jnp.zeros((16,), jnp.float32)
            return 0
        lax.fori_loop(0, rpt // 16, _za, 0)
        for p in range(NT):
            pltpu.sync_copy(
                partH.at[pl.ds(((c * 2 + hl) * NT + p) * NP + s * rpt, rpt)],
                rbuf_v)

            def _acc(g, _):
                acc_v[pl.ds(g * 16, 16)] = (acc_v[pl.ds(g * 16, 16)]
                                            + rbuf_v[pl.ds(g * 16, 16)])
                return 0
            lax.fori_loop(0, rpt // 16, _acc, 0)
        pltpu.sync_copy(acc_v,
                        finH.at[pl.ds((c * 2 + hl) * NP + s * rpt, rpt)])
    plsc.subcore_barrier()
    pltpu.sync_copy(finH.at[pl.ds((c * 2 + 0) * NP, NP)], den0)
    pltpu.sync_copy(finH.at[pl.ds((c * 2 + 1) * NP, NP)], den1)

    def _chunk_c(ch, _):
        off = s * ept + ch * ECH
        pltpu.sync_copy(edst.at[pl.ds(off, ECH)], didx_v)
        pltpu.sync_copy(ew.at[pl.ds(off, ECH)], ewv_v)
        for g in range(8):
            ewg = ewv_v[pl.ds(g * 16, 16)]
            di = didx_v[pl.ds(g * 16, 16)]
            for hl, (exb, den, cfb) in enumerate(((exb0, den0, cf0_v),
                                                  (exb1, den1, cf1_v))):
                dv = plsc.load_gather(den, [di])
                ex = exb[pl.ds(ch * ECH + g * 16, 16)]
                cfb[pl.ds(g * 16, 16)] = ex * ewg / (dv + 1e-16)
        hg = 2 * c
        pltpu.sync_copy(cf0_v, coefT.at[pl.ds(hg * EP + off, ECH)])
        pltpu.sync_copy(cf1_v, coefT.at[pl.ds((hg + 1) * EP + off, ECH)])
        return 0
    lax.fori_loop(0, nch, _chunk_c, 0)


def _p1(tabs, esrc, edst, ew, emaskf):
    ept = EP // NT
    rpt = NP // NT
    f32 = jnp.float32
    i32 = jnp.int32
    fn = pl.kernel(
        _p1_body,
        out_type=(jax.ShapeDtypeStruct((4 * EP,), f32),
                  jax.ShapeDtypeStruct((2 * 2 * NT * NP,), f32),
                  jax.ShapeDtypeStruct((2 * 2 * NP,), f32)),
        mesh=_sc_mesh(),
        compiler_params=_sc_params(),
        scratch_types=[
            pltpu.VMEM((NP,), f32), pltpu.VMEM((NP,), f32),
            pltpu.VMEM((NP,), f32), pltpu.VMEM((NP,), f32),
            pltpu.VMEM((NP,), f32), pltpu.VMEM((NP,), f32),
            pltpu.VMEM((ept,), f32), pltpu.VMEM((ept,), f32),
            pltpu.VMEM((ECH,), i32), pltpu.VMEM((ECH,), i32),
            pltpu.VMEM((ECH,), f32), pltpu.VMEM((ECH,), f32),
            pltpu.VMEM((ECH,), f32), pltpu.VMEM((ECH,), f32),
            pltpu.VMEM((rpt,), f32), pltpu.VMEM((rpt,), f32),
        ],
    )
    coefT, _, _ = fn(tabs, esrc, edst, ew, emaskf)
    return coefT


def _p3_body(hmflat, esrc, edst, coefT, aggflat,
             rows_v, sidx_v, didx_v, cb_v,
             rowsb_v, sidxb_v, didxb_v, cbb_v, outS, sga, sgb, ssa, ssb):
    ept = EP // NT
    nch = ept // ECH
    rpt = NP // NT
    c = lax.axis_index("c")
    s = lax.axis_index("s")

    for s_local in range(4):
        slab = 4 * c + s_local
        head = 2 * c + (s_local // 2)
        def _zb(i, _):
            for q in range(8):
                rows_v[i, pl.ds(q * 16, 16)] = jnp.zeros((16,), jnp.float32)
            return 0
        lax.fori_loop(0, ECH, _zb, 0)
        for j in range(rpt // ECH):
            pltpu.sync_copy(rows_v, outS.at[pl.ds(s * rpt + j * ECH, ECH)])
        plsc.subcore_barrier()

        def _load_idx(ch, sidx_v, didx_v, cb_v):
            off = s * ept + ch * ECH
            pltpu.sync_copy(esrc.at[pl.ds(off, ECH)], sidx_v)
            for g in range(8):
                sidx_v[pl.ds(g * 16, 16)] = (sidx_v[pl.ds(g * 16, 16)]
                                             + slab * NP)
            pltpu.sync_copy(edst.at[pl.ds(off, ECH)], didx_v)
            pltpu.sync_copy(coefT.at[pl.ds(head * EP + off, ECH)], cb_v)

        def _scale_all(rows_v, cb_v):
            def _scale(rr, _):
                for u in range(8):
                    r = rr * 8 + u
                    spl = plsc.load_gather(
                        cb_v, [jnp.full((16,), r, jnp.int32)])
                    for q in range(8):
                        rows_v[r, pl.ds(q * 16, 16)] = (
                            rows_v[r, pl.ds(q * 16, 16)] * spl)
                return 0
            lax.fori_loop(0, ECH // 8, _scale, 0)

        _load_idx(0, sidx_v, didx_v, cb_v)
        pltpu.async_copy(hmflat.at[sidx_v], rows_v, sga)

        def _pair(i, _):
            @pl.when(i > 0)
            def _():
                pltpu.make_async_copy(hmflat.at[pl.ds(0, ECH)],
                                      outS.at[pl.ds(0, ECH)], ssb).wait()
            _load_idx(2 * i + 1, sidxb_v, didxb_v, cbb_v)
            pltpu.async_copy(hmflat.at[sidxb_v], rowsb_v, sgb)
            pltpu.make_async_copy(hmflat.at[pl.ds(0, ECH)], rows_v, sga).wait()
            _scale_all(rows_v, cb_v)
            pltpu.async_copy(rows_v, outS.at[didx_v], ssa, add=True)
            pltpu.make_async_copy(hmflat.at[pl.ds(0, ECH)],
                                  rowsb_v, sgb).wait()
            _scale_all(rowsb_v, cbb_v)
            pltpu.make_async_copy(hmflat.at[pl.ds(0, ECH)],
                                  outS.at[pl.ds(0, ECH)], ssa).wait()
            nxt = jnp.minimum(2 * i + 2, nch - 1)
            _load_idx(nxt, sidx_v, didx_v, cb_v)
            pltpu.async_copy(hmflat.at[sidx_v], rows_v, sga)
            pltpu.async_copy(rowsb_v, outS.at[didxb_v], ssb, add=True)
            return 0
        lax.fori_loop(0, nch // 2, _pair, 0)
        pltpu.make_async_copy(hmflat.at[pl.ds(0, ECH)], rows_v, sga).wait()
        pltpu.make_async_copy(hmflat.at[pl.ds(0, ECH)],
                              outS.at[pl.ds(0, ECH)], ssb).wait()
        plsc.subcore_barrier()

        for j in range(rpt // ECH):
            pltpu.sync_copy(
                outS.at[pl.ds(s * rpt + j * ECH, ECH)],
                aggflat.at[pl.ds(slab * NP + s * rpt + j * ECH, ECH)])
        plsc.subcore_barrier()


def _p3(hmflat, esrc, edst, coefT):
    f32 = jnp.float32
    i32 = jnp.int32
    fn = pl.kernel(
        _p3_body,
        out_type=jax.ShapeDtypeStruct((8 * NP, 128), f32),
        mesh=_sc_mesh(),
        compiler_params=_sc_params(),
        scratch_types=[
            pltpu.VMEM((ECH, 128), f32),
            pltpu.VMEM((ECH,), i32), pltpu.VMEM((ECH,), i32),
            pltpu.VMEM((ECH,), f32),
            pltpu.VMEM((ECH, 128), f32), pltpu.VMEM((ECH,), i32),
            pltpu.VMEM((ECH,), i32), pltpu.VMEM((ECH,), f32),
            pltpu.VMEM_SHARED((NP, 128), f32),
            pltpu.SemaphoreType.DMA, pltpu.SemaphoreType.DMA,
            pltpu.SemaphoreType.DMA, pltpu.SemaphoreType.DMA,
        ],
    )
    return fn(hmflat, esrc, edst, coefT)


def _edge_phase(hm, esrc, edst, ew, emaskf, attn_panel):
    A8t = hm[attn_panel, :, 0:8].T
    tabs = jnp.stack([jnp.stack([A8t[0], A8t[1], A8t[4], A8t[5]]),
                      jnp.stack([A8t[2], A8t[3], A8t[6], A8t[7]])])
    coefT = _p1(tabs, esrc, edst, ew, emaskf)
    hmflat = hm.reshape(-1, 128)
    aggflat = _p3(hmflat, esrc, edst, coefT)
    return aggflat.reshape(8, NP, 128)


def _build_attn_mat(a_s, a_d):
    A = jnp.zeros((HC, 16), jnp.float32)
    hid = jnp.arange(HC) // HIDDEN
    pos = jnp.arange(HC) % HIDDEN
    As = a_s[hid, pos]
    Ad = a_d[hid, pos]
    onehot = (jnp.arange(16)[None, :] == hid[:, None]).astype(jnp.float32)
    onehot4 = (jnp.arange(16)[None, :] == (hid[:, None] + 4)).astype(jnp.float32)
    A = onehot * As[:, None] + onehot4 * Ad[:, None]
    return A


def _wcat(W, a_s, a_d, Wr=None):
    A = _build_attn_mat(a_s, a_d)
    Apad = jnp.pad(W @ A, ((0, 0), (0, 112)))
    parts = [W]
    if Wr is not None:
        parts.append(Wr)
    parts.append(Apad)
    return jnp.concatenate(parts, axis=1)


def kernel(x, bn_g, bn_b, bn_mean, bn_var, W1, as1, ad1, b1, W2, as2, ad2, b2,
           W3, as3, ad3, b3, W4, as4, ad4, b4, Wr1, br1, Wr2, br2, g1, be1,
           g2, be2, g3, be3, g4, be4, Wf, bf):
    xpad = jnp.pad(x, ((0, NP - N), (0, 0)))
    xb, xn = _prep(xpad, bn_g, bn_b, bn_mean, bn_var)
    nbr, ewk, hn = _simtopk(xn)
    nbr = nbr[:N]
    ewk = ewk[:N]
    missing = hn[:N - 1, 0] == 0

    base = jnp.arange(N - 1, dtype=jnp.int32)
    esrc = jnp.concatenate([
        jnp.repeat(jnp.arange(N, dtype=jnp.int32), K), base, base + 1,
        jnp.zeros((EP - N * K - 2 * (N - 1),), jnp.int32)])
    edst = jnp.concatenate([
        nbr.reshape(-1), base + 1, base,
        jnp.zeros((EP - N * K - 2 * (N - 1),), jnp.int32)])
    ew = jnp.concatenate([
        ewk.reshape(-1), jnp.full((2 * (N - 1),), TW, jnp.float32),
        jnp.zeros((EP - N * K - 2 * (N - 1),), jnp.float32)])
    missf = missing.astype(jnp.float32)
    emaskf = jnp.concatenate([
        jnp.ones((N * K,), jnp.float32), missf, missf,
        jnp.zeros((EP - N * K - 2 * (N - 1),), jnp.float32)])

    zero8 = jnp.zeros((8 * 128,), jnp.float32)
    zero128 = jnp.zeros((128,), jnp.float32)

    Wc = _wcat(W1, as1, ad1, Wr1)
    bias = jnp.concatenate([zero8, br1, zero128])
    hm = _mm(xb, Wc, bias)
    agg = _edge_phase(hm, esrc, edst, ew, emaskf, 16)
    rspec = pl.BlockSpec((8, RB, 128), lambda i: (1, i, 0))
    h = _post12(agg, hm, rspec, b1, g1, be1, r_slabbed=True)

    Wc = _wcat(W2, as2, ad2)
    bias = jnp.concatenate([zero8, zero128])
    hm = _mm(h, Wc, bias)
    agg = _edge_phase(hm, esrc, edst, ew, emaskf, 8)
    rspec = pl.BlockSpec((RB, HC), lambda i: (i, 0))
    h = _post12(agg, h, rspec, b2, g2, be2, r_slabbed=False)

    Wc = _wcat(W3, as3, ad3, Wr2)
    bias = jnp.concatenate([zero8, br2, zero128])
    hm = _mm(h, Wc, bias)
    agg = _edge_phase(hm, esrc, edst, ew, emaskf, 10)
    rspec = pl.BlockSpec((2, RB, 128), lambda i: (4, i, 0))
    wf_pad = jnp.pad(Wf, ((0, 0), (0, 128 - NUM_CLASSES)))
    bf_pad = jnp.pad(bf, (0, 128 - NUM_CLASSES))[None]
    h = _post34(agg, hm, rspec, b3, g3, be3, True, False, wf_pad, bf_pad)

    Wc = _wcat(W4, as4, ad4)
    bias = jnp.concatenate([zero8, zero128])
    hm = _mm(h, Wc, bias)
    agg = _edge_phase(hm, esrc, edst, ew, emaskf, 8)
    rspec = pl.BlockSpec((RB, HIDDEN), lambda i: (i, 0))
    out = _post34(agg, h, rspec, b4, g4, be4, False, True, wf_pad, bf_pad)

    return out[:N, :NUM_CLASSES]


# speedup vs baseline: 11.7722x; 1.5477x over previous
import functools

import jax
import jax.numpy as jnp
from jax import lax
from jax.experimental import pallas as pl
from jax.experimental.pallas import tpu as pltpu
from jax.experimental.pallas import tpu_sc as plsc

N = 10000
EMBED = 128
HIDDEN = 256
HEADS = 4
HC = HIDDEN * HEADS
NUM_CLASSES = 64
K = 8
TW = 1.0

NP = 10240
RB = 256
CB = 512
EP = 102400
NEG = -1e30


def _prep_body(x_ref, g_ref, b_ref, m_ref, v_ref, xb_ref, xn_ref):
    x = x_ref[...]
    xb = (x - m_ref[...]) / jnp.sqrt(v_ref[...] + 1e-5) * g_ref[...] + b_ref[...]
    nrm = jnp.sqrt((xb * xb).sum(axis=1, keepdims=True))
    xb_ref[...] = xb
    xn_ref[...] = xb / (nrm + 1e-8)


def _prep(x, bn_g, bn_b, bn_mean, bn_var):
    grid = (NP // RB,)
    return pl.pallas_call(
        _prep_body,
        grid=grid,
        in_specs=[pl.BlockSpec((RB, EMBED), lambda i: (i, 0))] +
                 [pl.BlockSpec((1, EMBED), lambda i: (0, 0))] * 4,
        out_specs=[pl.BlockSpec((RB, EMBED), lambda i: (i, 0))] * 2,
        out_shape=[jax.ShapeDtypeStruct((NP, EMBED), jnp.float32)] * 2,
    )(x, bn_g[None], bn_b[None], bn_mean[None], bn_var[None])


def _simtopk_body(xr_ref, xc_ref, nbr_ref, ew_ref, hn_ref, cv_ref, cg_ref):
    i = pl.program_id(0)
    j = pl.program_id(1)
    W = 128 + CB

    @pl.when(j == 0)
    def _init():
        cv_ref[...] = jnp.full((RB, 128), NEG, jnp.float32)
        cg_ref[...] = jnp.zeros((RB, 128), jnp.float32)

    s = lax.dot_general(xr_ref[...], xc_ref[...], (((1,), (1,)), ((), ())),
                        preferred_element_type=jnp.float32)
    rowid = (jnp.float32(i * RB) +
             lax.broadcasted_iota(jnp.int32, (RB, CB), 0).astype(jnp.float32))
    colid = (jnp.float32(j * CB) +
             lax.broadcasted_iota(jnp.int32, (RB, CB), 1).astype(jnp.float32))
    s = jnp.where((colid == rowid) | (colid >= jnp.float32(N)), NEG, s)

    lanes = lax.broadcasted_iota(jnp.int32, (RB, W), 1).astype(jnp.float32)
    vb = jnp.concatenate([cv_ref[...], s], axis=1)
    gb = jnp.concatenate([cg_ref[...], colid], axis=1)
    ms, gs = [], []
    for _ in range(K):
        m = jnp.max(vb, axis=1)[:, None]
        keyloc = jnp.where(vb == m, lanes, jnp.float32(1e9))
        ni = jnp.min(keyloc, axis=1)[:, None]
        hit = keyloc == ni
        gs.append(jnp.sum(jnp.where(hit, gb, 0.0), axis=1))
        ms.append(m[:, 0])
        vb = jnp.where(hit, NEG, vb)
    lane8 = lax.broadcasted_iota(jnp.int32, (RB, 128), 1).astype(jnp.float32)
    cv = jnp.full((RB, 128), NEG, jnp.float32)
    cg = jnp.zeros((RB, 128), jnp.float32)
    for t in range(K):
        sel = lane8 == jnp.float32(t)
        cv = jnp.where(sel, ms[t][:, None], cv)
        cg = jnp.where(sel, gs[t][:, None], cg)
    cv_ref[...] = cv
    cg_ref[...] = cg

    @pl.when(j == (NP // CB) - 1)
    def _fin():
        nbf = cg_ref[:, 0:K]
        vv = cv_ref[:, 0:K]
        rid = (jnp.float32(i * RB)
               + lax.broadcasted_iota(jnp.int32, (RB, K), 0)
               .astype(jnp.float32))
        nbr_ref[...] = nbf.astype(jnp.int32)
        ew_ref[...] = vv + TW * (jnp.abs(nbf - rid) == 1).astype(jnp.float32)
        hn = (nbf == rid + 1).any(axis=1, keepdims=True)
        hn_ref[...] = jnp.broadcast_to(hn, (RB, K)).astype(jnp.int32)


def _simtopk(xn):
    grid = (NP // RB, NP // CB)
    return pl.pallas_call(
        _simtopk_body,
        grid=grid,
        in_specs=[pl.BlockSpec((RB, EMBED), lambda i, j: (i, 0)),
                  pl.BlockSpec((CB, EMBED), lambda i, j: (j, 0))],
        out_specs=[pl.BlockSpec((RB, K), lambda i, j: (i, 0))] * 3,
        out_shape=[jax.ShapeDtypeStruct((NP, K), jnp.int32),
                   jax.ShapeDtypeStruct((NP, K), jnp.float32),
                   jax.ShapeDtypeStruct((NP, K), jnp.int32)],
        scratch_shapes=[pltpu.VMEM((RB, 128), jnp.float32),
                        pltpu.VMEM((RB, 128), jnp.float32)],
    )(xn, xn)


def _mm_body(x_ref, w_ref, b_ref, o_ref):
    o_ref[0] = jnp.dot(x_ref[...], w_ref[...],
                       preferred_element_type=jnp.float32) + b_ref[0]


def _mm(x, Wcat, bias):
    Pn = Wcat.shape[1] // 128
    Kd = x.shape[1]
    grid = (NP // RB, Pn)
    return pl.pallas_call(
        _mm_body,
        grid=grid,
        in_specs=[pl.BlockSpec((RB, Kd), lambda i, j: (i, 0)),
                  pl.BlockSpec((Kd, 128), lambda i, j: (0, j)),
                  pl.BlockSpec((1, 1, 128), lambda i, j: (j, 0, 0))],
        out_specs=pl.BlockSpec((1, RB, 128), lambda i, j: (j, i, 0)),
        out_shape=jax.ShapeDtypeStruct((Pn, NP, 128), jnp.float32),
    )(x, Wcat, bias.reshape(Pn, 1, 128))


def _post12_body(agg_ref, r_ref, b_ref, g_ref, be_ref, o_ref, *, r_slabbed):
    vs = []
    ssum = 0.0
    ssq = 0.0
    for q in range(8):
        r = r_ref[q] if r_slabbed else r_ref[:, q * 128:(q + 1) * 128]
        v = jnp.maximum(agg_ref[q] + b_ref[q], 0.0) + r
        vs.append(v)
        ssum = ssum + v.sum(axis=1, keepdims=True)
        ssq = ssq + (v * v).sum(axis=1, keepdims=True)
    mu = ssum / HC
    var = ssq / HC - mu * mu
    rstd = lax.rsqrt(var + 1e-5)
    for q in range(8):
        o_ref[:, q * 128:(q + 1) * 128] = (vs[q] - mu) * rstd * g_ref[q] + be_ref[q]


def _post12(agg, r, r_spec, b, g, be, r_slabbed):
    grid = (NP // RB,)
    return pl.pallas_call(
        functools.partial(_post12_body, r_slabbed=r_slabbed),
        grid=grid,
        in_specs=[pl.BlockSpec((8, RB, 128), lambda i: (0, i, 0)),
                  r_spec,
                  pl.BlockSpec((8, 1, 128), lambda i: (0, 0, 0)),
                  pl.BlockSpec((8, 1, 128), lambda i: (0, 0, 0)),
                  pl.BlockSpec((8, 1, 128), lambda i: (0, 0, 0))],
        out_specs=pl.BlockSpec((RB, HC), lambda i: (i, 0)),
        out_shape=jax.ShapeDtypeStruct((NP, HC), jnp.float32),
    )(agg, r, b.reshape(8, 1, 128), g.reshape(8, 1, 128), be.reshape(8, 1, 128))


def _post34_body(agg_ref, r_ref, b_ref, g_ref, be_ref, wf_ref, bf_ref, o_ref,
                 *, r_slabbed, final):
    vs = []
    ssum = 0.0
    ssq = 0.0
    for p in range(2):
        m = 0.25 * (agg_ref[p] + agg_ref[2 + p] + agg_ref[4 + p] + agg_ref[6 + p])
        r = r_ref[p] if r_slabbed else r_ref[:, p * 128:(p + 1) * 128]
        v = jnp.maximum(m + b_ref[p], 0.0) + r
        vs.append(v)
        ssum = ssum + v.sum(axis=1, keepdims=True)
        ssq = ssq + (v * v).sum(axis=1, keepdims=True)
    mu = ssum / HIDDEN
    var = ssq / HIDDEN - mu * mu
    rstd = lax.rsqrt(var + 1e-5)
    hs = [(vs[p] - mu) * rstd * g_ref[p] + be_ref[p] for p in range(2)]
    if final:
        acc = jnp.dot(hs[0], wf_ref[0:128], preferred_element_type=jnp.float32)
        acc = acc + jnp.dot(hs[1], wf_ref[128:256],
                            preferred_element_type=jnp.float32)
        o_ref[...] = acc + bf_ref[...]
    else:
        o_ref[:, 0:128] = hs[0]
        o_ref[:, 128:256] = hs[1]


def _post34(agg, r, r_spec, b, g, be, r_slabbed, final, wf_pad, bf_pad):
    grid = (NP // RB,)
    ocols = 128 if final else HIDDEN
    return pl.pallas_call(
        functools.partial(_post34_body, r_slabbed=r_slabbed, final=final),
        grid=grid,
        in_specs=[pl.BlockSpec((8, RB, 128), lambda i: (0, i, 0)),
                  r_spec,
                  pl.BlockSpec((2, 1, 128), lambda i: (0, 0, 0)),
                  pl.BlockSpec((2, 1, 128), lambda i: (0, 0, 0)),
                  pl.BlockSpec((2, 1, 128), lambda i: (0, 0, 0)),
                  pl.BlockSpec((HIDDEN, 128), lambda i: (0, 0)),
                  pl.BlockSpec((1, 128), lambda i: (0, 0))],
        out_specs=pl.BlockSpec((RB, ocols), lambda i: (i, 0)),
        out_shape=jax.ShapeDtypeStruct((NP, ocols), jnp.float32),
    )(agg, r, b.reshape(2, 1, 128), g.reshape(2, 1, 128),
      be.reshape(2, 1, 128), wf_pad, bf_pad)


NT = 16
ECH = 128
ROWS_PT = None


def _sc_mesh():
    return plsc.VectorSubcoreMesh(core_axis_name="c", subcore_axis_name="s")


def _sc_params():
    return pltpu.CompilerParams(needs_layout_passes=False)


def _i16():
    return lax.iota(jnp.int32, 16)


def _p1_body(tabs, esrc, edst, ew, emaskf, coefT, partH, finH,
             t0, t1, t2, t3, den0, den1, exb0, exb1, sidx_v, didx_v, mk_v,
             ewv_v, cf0_v, cf1_v, rbuf_v, acc_v):
    ept = EP // NT
    nch = ept // ECH
    rpt = NP // NT
    c = lax.axis_index("c")
    s = lax.axis_index("s")
    lane = _i16()

    pltpu.sync_copy(tabs.at[c].at[0], t0)
    pltpu.sync_copy(tabs.at[c].at[1], t1)
    pltpu.sync_copy(tabs.at[c].at[2], t2)
    pltpu.sync_copy(tabs.at[c].at[3], t3)

    def _zd(i, _):
        den0[pl.ds(i * 16, 16)] = jnp.zeros((16,), jnp.float32)
        den1[pl.ds(i * 16, 16)] = jnp.zeros((16,), jnp.float32)
        return 0
    lax.fori_loop(0, NP // 16, _zd, 0)

    def _chunk_b(ch, _):
        off = s * ept + ch * ECH
        pltpu.sync_copy(esrc.at[pl.ds(off, ECH)], sidx_v)
        pltpu.sync_copy(edst.at[pl.ds(off, ECH)], didx_v)
        pltpu.sync_copy(emaskf.at[pl.ds(off, ECH)], mk_v)
        for g in range(8):
            si = sidx_v[pl.ds(g * 16, 16)]
            di = didx_v[pl.ds(g * 16, 16)]
            mk = mk_v[pl.ds(g * 16, 16)]
            for hl, (ta, td, exb, den) in enumerate(((t0, t2, exb0, den0),
                                                     (t1, t3, exb1, den1))):
                e = plsc.load_gather(ta, [si]) + plsc.load_gather(td, [di])
                e = jnp.where(e > 0.0, e, 0.2 * e)
                ex = jnp.exp(e) * mk
                exb[pl.ds(ch * ECH + g * 16, 16)] = ex
                for l in range(16):
                    plsc.addupdate_scatter(den, [di], ex, mask=lane == l)
        return 0
    lax.fori_loop(0, nch, _chunk_b, 0)

    pltpu.sync_copy(den0, partH.at[pl.ds(((c * 2 + 0) * NT + s) * NP, NP)])
    pltpu.sync_copy(den1, partH.at[pl.ds(((c * 2 + 1) * NT + s) * NP, NP)])
    plsc.subcore_barrier()
    for hl, den in ((0, den0), (1, den1)):
        def _za(i, _):
            acc_v[pl.ds(i * 16, 16)] = jnp.zeros((16,), jnp.float32)
            return 0
        lax.fori_loop(0, rpt // 16, _za, 0)
        for p in range(NT):
            pltpu.sync_copy(
                partH.at[pl.ds(((c * 2 + hl) * NT + p) * NP + s * rpt, rpt)],
                rbuf_v)

            def _acc(g, _):
                acc_v[pl.ds(g * 16, 16)] = (acc_v[pl.ds(g * 16, 16)]
                                            + rbuf_v[pl.ds(g * 16, 16)])
                return 0
            lax.fori_loop(0, rpt // 16, _acc, 0)
        pltpu.sync_copy(acc_v,
                        finH.at[pl.ds((c * 2 + hl) * NP + s * rpt, rpt)])
    plsc.subcore_barrier()
    pltpu.sync_copy(finH.at[pl.ds((c * 2 + 0) * NP, NP)], den0)
    pltpu.sync_copy(finH.at[pl.ds((c * 2 + 1) * NP, NP)], den1)

    def _chunk_c(ch, _):
        off = s * ept + ch * ECH
        pltpu.sync_copy(edst.at[pl.ds(off, ECH)], didx_v)
        pltpu.sync_copy(ew.at[pl.ds(off, ECH)], ewv_v)
        for g in range(8):
            ewg = ewv_v[pl.ds(g * 16, 16)]
            di = didx_v[pl.ds(g * 16, 16)]
            for hl, (exb, den, cfb) in enumerate(((exb0, den0, cf0_v),
                                                  (exb1, den1, cf1_v))):
                dv = plsc.load_gather(den, [di])
                ex = exb[pl.ds(ch * ECH + g * 16, 16)]
                cfb[pl.ds(g * 16, 16)] = ex * ewg / (dv + 1e-16)
        hg = 2 * c
        pltpu.sync_copy(cf0_v, coefT.at[pl.ds(hg * EP + off, ECH)])
        pltpu.sync_copy(cf1_v, coefT.at[pl.ds((hg + 1) * EP + off, ECH)])
        return 0
    lax.fori_loop(0, nch, _chunk_c, 0)


def _p1(tabs, esrc, edst, ew, emaskf):
    ept = EP // NT
    rpt = NP // NT
    f32 = jnp.float32
    i32 = jnp.int32
    fn = pl.kernel(
        _p1_body,
        out_type=(jax.ShapeDtypeStruct((4 * EP,), f32),
                  jax.ShapeDtypeStruct((2 * 2 * NT * NP,), f32),
                  jax.ShapeDtypeStruct((2 * 2 * NP,), f32)),
        mesh=_sc_mesh(),
        compiler_params=_sc_params(),
        scratch_types=[
            pltpu.VMEM((NP,), f32), pltpu.VMEM((NP,), f32),
            pltpu.VMEM((NP,), f32), pltpu.VMEM((NP,), f32),
            pltpu.VMEM((NP,), f32), pltpu.VMEM((NP,), f32),
            pltpu.VMEM((ept,), f32), pltpu.VMEM((ept,), f32),
            pltpu.VMEM((ECH,), i32), pltpu.VMEM((ECH,), i32),
            pltpu.VMEM((ECH,), f32), pltpu.VMEM((ECH,), f32),
            pltpu.VMEM((ECH,), f32), pltpu.VMEM((ECH,), f32),
            pltpu.VMEM((rpt,), f32), pltpu.VMEM((rpt,), f32),
        ],
    )
    coefT, _, _ = fn(tabs, esrc, edst, ew, emaskf)
    return coefT


def _p3_body(hmflat, esrc, edst, coefT, aggflat,
             rows_v, sidx_v, didx_v, cb_v,
             rowsb_v, sidxb_v, didxb_v, cbb_v, outS, sga, sgb, ssa, ssb):
    ept = EP // NT
    nch = ept // ECH
    rpt = NP // NT
    c = lax.axis_index("c")
    s = lax.axis_index("s")

    for s_local in range(4):
        slab = 4 * c + s_local
        head = 2 * c + (s_local // 2)
        def _zb(i, _):
            for q in range(8):
                rows_v[i, pl.ds(q * 16, 16)] = jnp.zeros((16,), jnp.float32)
            return 0
        lax.fori_loop(0, ECH, _zb, 0)
        for j in range(rpt // ECH):
            pltpu.sync_copy(rows_v, outS.at[pl.ds(s * rpt + j * ECH, ECH)])
        plsc.subcore_barrier()

        def _load_idx(ch, sidx_v, didx_v, cb_v):
            off = s * ept + ch * ECH
            pltpu.sync_copy(esrc.at[pl.ds(off, ECH)], sidx_v)
            for g in range(8):
                sidx_v[pl.ds(g * 16, 16)] = (sidx_v[pl.ds(g * 16, 16)]
                                             + slab * NP)
            pltpu.sync_copy(edst.at[pl.ds(off, ECH)], didx_v)
            pltpu.sync_copy(coefT.at[pl.ds(head * EP + off, ECH)], cb_v)

        def _scale_all(rows_v, cb_v):
            def _scale(rr, _):
                for u in range(8):
                    r = rr * 8 + u
                    spl = plsc.load_gather(
                        cb_v, [jnp.full((16,), r, jnp.int32)])
                    for q in range(8):
                        rows_v[r, pl.ds(q * 16, 16)] = (
                            rows_v[r, pl.ds(q * 16, 16)] * spl)
                return 0
            lax.fori_loop(0, ECH // 8, _scale, 0)

        _load_idx(0, sidx_v, didx_v, cb_v)
        pltpu.async_copy(hmflat.at[sidx_v], rows_v, sga)

        def _pair(i, _):
            @pl.when(i > 0)
            def _():
                pltpu.make_async_copy(hmflat.at[pl.ds(0, ECH)],
                                      outS.at[pl.ds(0, ECH)], ssb).wait()
            _load_idx(2 * i + 1, sidxb_v, didxb_v, cbb_v)
            pltpu.async_copy(hmflat.at[sidxb_v], rowsb_v, sgb)
            pltpu.make_async_copy(hmflat.at[pl.ds(0, ECH)], rows_v, sga).wait()
            _scale_all(rows_v, cb_v)
            pltpu.async_copy(rows_v, outS.at[didx_v], ssa, add=True)
            pltpu.make_async_copy(hmflat.at[pl.ds(0, ECH)],
                                  rowsb_v, sgb).wait()
            _scale_all(rowsb_v, cbb_v)
            pltpu.make_async_copy(hmflat.at[pl.ds(0, ECH)],
                                  outS.at[pl.ds(0, ECH)], ssa).wait()
            nxt = jnp.minimum(2 * i + 2, nch - 1)
            _load_idx(nxt, sidx_v, didx_v, cb_v)
            pltpu.async_copy(hmflat.at[sidx_v], rows_v, sga)
            pltpu.async_copy(rowsb_v, outS.at[didxb_v], ssb, add=True)
            return 0
        lax.fori_loop(0, nch // 2, _pair, 0)
        pltpu.make_async_copy(hmflat.at[pl.ds(0, ECH)], rows_v, sga).wait()
        pltpu.make_async_copy(hmflat.at[pl.ds(0, ECH)],
                              outS.at[pl.ds(0, ECH)], ssb).wait()
        plsc.subcore_barrier()

        for j in range(rpt // ECH):
            pltpu.sync_copy(
                outS.at[pl.ds(s * rpt + j * ECH, ECH)],
                aggflat.at[pl.ds(slab * NP + s * rpt + j * ECH, ECH)])
        plsc.subcore_barrier()


def _p3(hmflat, esrc, edst, coefT):
    f32 = jnp.float32
    i32 = jnp.int32
    fn = pl.kernel(
        _p3_body,
        out_type=jax.ShapeDtypeStruct((8 * NP, 128), f32),
        mesh=_sc_mesh(),
        compiler_params=_sc_params(),
        scratch_types=[
            pltpu.VMEM((ECH, 128), f32),
            pltpu.VMEM((ECH,), i32), pltpu.VMEM((ECH,), i32),
            pltpu.VMEM((ECH,), f32),
            pltpu.VMEM((ECH, 128), f32), pltpu.VMEM((ECH,), i32),
            pltpu.VMEM((ECH,), i32), pltpu.VMEM((ECH,), f32),
            pltpu.VMEM_SHARED((NP, 128), f32),
            pltpu.SemaphoreType.DMA, pltpu.SemaphoreType.DMA,
            pltpu.SemaphoreType.DMA, pltpu.SemaphoreType.DMA,
        ],
    )
    return fn(hmflat, esrc, edst, coefT)


def _edge_phase(hm, esrc, edst, ew, emaskf, attn_panel):
    A8t = hm[attn_panel, :, 0:8].T
    tabs = jnp.stack([jnp.stack([A8t[0], A8t[1], A8t[4], A8t[5]]),
                      jnp.stack([A8t[2], A8t[3], A8t[6], A8t[7]])])
    coefT = _p1(tabs, esrc, edst, ew, emaskf)
    hmflat = hm.reshape(-1, 128)
    aggflat = _p3(hmflat, esrc, edst, coefT)
    return aggflat.reshape(8, NP, 128)


def _build_attn_mat(a_s, a_d):
    A = jnp.zeros((HC, 16), jnp.float32)
    hid = jnp.arange(HC) // HIDDEN
    pos = jnp.arange(HC) % HIDDEN
    As = a_s[hid, pos]
    Ad = a_d[hid, pos]
    onehot = (jnp.arange(16)[None, :] == hid[:, None]).astype(jnp.float32)
    onehot4 = (jnp.arange(16)[None, :] == (hid[:, None] + 4)).astype(jnp.float32)
    A = onehot * As[:, None] + onehot4 * Ad[:, None]
    return A


def _wcat(W, a_s, a_d, Wr=None):
    A = _build_attn_mat(a_s, a_d)
    Apad = jnp.pad(W @ A, ((0, 0), (0, 112)))
    parts = [W]
    if Wr is not None:
        parts.append(Wr)
    parts.append(Apad)
    return jnp.concatenate(parts, axis=1)


def kernel(x, bn_g, bn_b, bn_mean, bn_var, W1, as1, ad1, b1, W2, as2, ad2, b2,
           W3, as3, ad3, b3, W4, as4, ad4, b4, Wr1, br1, Wr2, br2, g1, be1,
           g2, be2, g3, be3, g4, be4, Wf, bf):
    xpad = jnp.pad(x, ((0, NP - N), (0, 0)))
    xb, xn = _prep(xpad, bn_g, bn_b, bn_mean, bn_var)
    nbr, ewk, hn = _simtopk(xn)
    nbr = nbr[:N]
    ewk = ewk[:N]
    missing = hn[:N - 1, 0] == 0

    base = jnp.arange(N - 1, dtype=jnp.int32)
    esrc = jnp.concatenate([
        jnp.repeat(jnp.arange(N, dtype=jnp.int32), K), base, base + 1,
        jnp.zeros((EP - N * K - 2 * (N - 1),), jnp.int32)])
    edst = jnp.concatenate([
        nbr.reshape(-1), base + 1, base,
        jnp.zeros((EP - N * K - 2 * (N - 1),), jnp.int32)])
    ew = jnp.concatenate([
        ewk.reshape(-1), jnp.full((2 * (N - 1),), TW, jnp.float32),
        jnp.zeros((EP - N * K - 2 * (N - 1),), jnp.float32)])
    missf = missing.astype(jnp.float32)
    emaskf = jnp.concatenate([
        jnp.ones((N * K,), jnp.float32), missf, missf,
        jnp.zeros((EP - N * K - 2 * (N - 1),), jnp.float32)])

    zero8 = jnp.zeros((8 * 128,), jnp.float32)
    zero128 = jnp.zeros((128,), jnp.float32)

    Wc = _wcat(W1, as1, ad1, Wr1)
    bias = jnp.concatenate([zero8, br1, zero128])
    hm = _mm(xb, Wc, bias)
    agg = _edge_phase(hm, esrc, edst, ew, emaskf, 16)
    rspec = pl.BlockSpec((8, RB, 128), lambda i: (1, i, 0))
    h = _post12(agg, hm, rspec, b1, g1, be1, r_slabbed=True)

    Wc = _wcat(W2, as2, ad2)
    bias = jnp.concatenate([zero8, zero128])
    hm = _mm(h, Wc, bias)
    agg = _edge_phase(hm, esrc, edst, ew, emaskf, 8)
    rspec = pl.BlockSpec((RB, HC), lambda i: (i, 0))
    h = _post12(agg, h, rspec, b2, g2, be2, r_slabbed=False)

    Wc = _wcat(W3, as3, ad3, Wr2)
    bias = jnp.concatenate([zero8, br2, zero128])
    hm = _mm(h, Wc, bias)
    agg = _edge_phase(hm, esrc, edst, ew, emaskf, 10)
    rspec = pl.BlockSpec((2, RB, 128), lambda i: (4, i, 0))
    wf_pad = jnp.pad(Wf, ((0, 0), (0, 128 - NUM_CLASSES)))
    bf_pad = jnp.pad(bf, (0, 128 - NUM_CLASSES))[None]
    h = _post34(agg, hm, rspec, b3, g3, be3, True, False, wf_pad, bf_pad)

    Wc = _wcat(W4, as4, ad4)
    bias = jnp.concatenate([zero8, zero128])
    hm = _mm(h, Wc, bias)
    agg = _edge_phase(hm, esrc, edst, ew, emaskf, 8)
    rspec = pl.BlockSpec((RB, HIDDEN), lambda i: (i, 0))
    out = _post34(agg, h, rspec, b4, g4, be4, False, True, wf_pad, bf_pad)

    return out[:N, :NUM_CLASSES]


# speedup vs baseline: 12.0557x; 1.0241x over previous
import functools

import jax
import jax.numpy as jnp
from jax import lax
from jax.experimental import pallas as pl
from jax.experimental.pallas import tpu as pltpu
from jax.experimental.pallas import tpu_sc as plsc

N = 10000
EMBED = 128
HIDDEN = 256
HEADS = 4
HC = HIDDEN * HEADS
NUM_CLASSES = 64
K = 8
TW = 1.0

NP = 10240
RB = 256
CB = 512
EP = 102400
NEG = -1e30


def _prep_body(x_ref, g_ref, b_ref, m_ref, v_ref, xb_ref, xn_ref):
    x = x_ref[...]
    xb = (x - m_ref[...]) / jnp.sqrt(v_ref[...] + 1e-5) * g_ref[...] + b_ref[...]
    nrm = jnp.sqrt((xb * xb).sum(axis=1, keepdims=True))
    xb_ref[...] = xb
    xn_ref[...] = xb / (nrm + 1e-8)


def _prep(x, bn_g, bn_b, bn_mean, bn_var):
    grid = (NP // RB,)
    return pl.pallas_call(
        _prep_body,
        grid=grid,
        in_specs=[pl.BlockSpec((RB, EMBED), lambda i: (i, 0))] +
                 [pl.BlockSpec((1, EMBED), lambda i: (0, 0))] * 4,
        out_specs=[pl.BlockSpec((RB, EMBED), lambda i: (i, 0))] * 2,
        out_shape=[jax.ShapeDtypeStruct((NP, EMBED), jnp.float32)] * 2,
    )(x, bn_g[None], bn_b[None], bn_mean[None], bn_var[None])


def _simtopk_body(xr_ref, xc_ref, nbr_ref, ew_ref, hn_ref, cv_ref, cg_ref):
    i = pl.program_id(0)
    j = pl.program_id(1)
    W = 128 + CB

    @pl.when(j == 0)
    def _init():
        cv_ref[...] = jnp.full((RB, 128), NEG, jnp.float32)
        cg_ref[...] = jnp.zeros((RB, 128), jnp.float32)

    s = lax.dot_general(xr_ref[...], xc_ref[...], (((1,), (1,)), ((), ())),
                        preferred_element_type=jnp.float32)
    rowid = (jnp.float32(i * RB) +
             lax.broadcasted_iota(jnp.int32, (RB, CB), 0).astype(jnp.float32))
    colid = (jnp.float32(j * CB) +
             lax.broadcasted_iota(jnp.int32, (RB, CB), 1).astype(jnp.float32))
    s = jnp.where((colid == rowid) | (colid >= jnp.float32(N)), NEG, s)

    lanes = lax.broadcasted_iota(jnp.int32, (RB, W), 1).astype(jnp.float32)
    vb = jnp.concatenate([cv_ref[...], s], axis=1)
    gb = jnp.concatenate([cg_ref[...], colid], axis=1)
    ms, gs = [], []
    for _ in range(K):
        m = jnp.max(vb, axis=1)[:, None]
        keyloc = jnp.where(vb == m, lanes, jnp.float32(1e9))
        ni = jnp.min(keyloc, axis=1)[:, None]
        hit = keyloc == ni
        gs.append(jnp.sum(jnp.where(hit, gb, 0.0), axis=1))
        ms.append(m[:, 0])
        vb = jnp.where(hit, NEG, vb)
    lane8 = lax.broadcasted_iota(jnp.int32, (RB, 128), 1).astype(jnp.float32)
    cv = jnp.full((RB, 128), NEG, jnp.float32)
    cg = jnp.zeros((RB, 128), jnp.float32)
    for t in range(K):
        sel = lane8 == jnp.float32(t)
        cv = jnp.where(sel, ms[t][:, None], cv)
        cg = jnp.where(sel, gs[t][:, None], cg)
    cv_ref[...] = cv
    cg_ref[...] = cg

    @pl.when(j == (NP // CB) - 1)
    def _fin():
        nbf = cg_ref[:, 0:K]
        vv = cv_ref[:, 0:K]
        rid = (jnp.float32(i * RB)
               + lax.broadcasted_iota(jnp.int32, (RB, K), 0)
               .astype(jnp.float32))
        nbr_ref[...] = nbf.astype(jnp.int32)
        ew_ref[...] = vv + TW * (jnp.abs(nbf - rid) == 1).astype(jnp.float32)
        hn = (nbf == rid + 1).any(axis=1, keepdims=True)
        hn_ref[...] = jnp.broadcast_to(hn, (RB, K)).astype(jnp.int32)


def _simtopk(xn):
    grid = (NP // RB, NP // CB)
    return pl.pallas_call(
        _simtopk_body,
        grid=grid,
        in_specs=[pl.BlockSpec((RB, EMBED), lambda i, j: (i, 0)),
                  pl.BlockSpec((CB, EMBED), lambda i, j: (j, 0))],
        out_specs=[pl.BlockSpec((RB, K), lambda i, j: (i, 0))] * 3,
        out_shape=[jax.ShapeDtypeStruct((NP, K), jnp.int32),
                   jax.ShapeDtypeStruct((NP, K), jnp.float32),
                   jax.ShapeDtypeStruct((NP, K), jnp.int32)],
        scratch_shapes=[pltpu.VMEM((RB, 128), jnp.float32),
                        pltpu.VMEM((RB, 128), jnp.float32)],
    )(xn, xn)


def _mm_body(x_ref, w_ref, b_ref, o_ref):
    o_ref[0] = jnp.dot(x_ref[...], w_ref[...],
                       preferred_element_type=jnp.float32) + b_ref[0]


def _mm(x, Wcat, bias):
    Pn = Wcat.shape[1] // 128
    Kd = x.shape[1]
    grid = (NP // RB, Pn)
    return pl.pallas_call(
        _mm_body,
        grid=grid,
        in_specs=[pl.BlockSpec((RB, Kd), lambda i, j: (i, 0)),
                  pl.BlockSpec((Kd, 128), lambda i, j: (0, j)),
                  pl.BlockSpec((1, 1, 128), lambda i, j: (j, 0, 0))],
        out_specs=pl.BlockSpec((1, RB, 128), lambda i, j: (j, i, 0)),
        out_shape=jax.ShapeDtypeStruct((Pn, NP, 128), jnp.float32),
    )(x, Wcat, bias.reshape(Pn, 1, 128))


def _post12_body(agg_ref, r_ref, b_ref, g_ref, be_ref, o_ref, *, r_slabbed):
    vs = []
    ssum = 0.0
    ssq = 0.0
    for q in range(8):
        r = r_ref[q] if r_slabbed else r_ref[:, q * 128:(q + 1) * 128]
        v = jnp.maximum(agg_ref[q] + b_ref[q], 0.0) + r
        vs.append(v)
        ssum = ssum + v.sum(axis=1, keepdims=True)
        ssq = ssq + (v * v).sum(axis=1, keepdims=True)
    mu = ssum / HC
    var = ssq / HC - mu * mu
    rstd = lax.rsqrt(var + 1e-5)
    for q in range(8):
        o_ref[:, q * 128:(q + 1) * 128] = (vs[q] - mu) * rstd * g_ref[q] + be_ref[q]


def _post12(agg, r, r_spec, b, g, be, r_slabbed):
    grid = (NP // RB,)
    return pl.pallas_call(
        functools.partial(_post12_body, r_slabbed=r_slabbed),
        grid=grid,
        in_specs=[pl.BlockSpec((8, RB, 128), lambda i: (0, i, 0)),
                  r_spec,
                  pl.BlockSpec((8, 1, 128), lambda i: (0, 0, 0)),
                  pl.BlockSpec((8, 1, 128), lambda i: (0, 0, 0)),
                  pl.BlockSpec((8, 1, 128), lambda i: (0, 0, 0))],
        out_specs=pl.BlockSpec((RB, HC), lambda i: (i, 0)),
        out_shape=jax.ShapeDtypeStruct((NP, HC), jnp.float32),
    )(agg, r, b.reshape(8, 1, 128), g.reshape(8, 1, 128), be.reshape(8, 1, 128))


def _post34_body(agg_ref, r_ref, b_ref, g_ref, be_ref, wf_ref, bf_ref, o_ref,
                 *, r_slabbed, final):
    vs = []
    ssum = 0.0
    ssq = 0.0
    for p in range(2):
        m = 0.25 * (agg_ref[p] + agg_ref[2 + p] + agg_ref[4 + p] + agg_ref[6 + p])
        r = r_ref[p] if r_slabbed else r_ref[:, p * 128:(p + 1) * 128]
        v = jnp.maximum(m + b_ref[p], 0.0) + r
        vs.append(v)
        ssum = ssum + v.sum(axis=1, keepdims=True)
        ssq = ssq + (v * v).sum(axis=1, keepdims=True)
    mu = ssum / HIDDEN
    var = ssq / HIDDEN - mu * mu
    rstd = lax.rsqrt(var + 1e-5)
    hs = [(vs[p] - mu) * rstd * g_ref[p] + be_ref[p] for p in range(2)]
    if final:
        acc = jnp.dot(hs[0], wf_ref[0:128], preferred_element_type=jnp.float32)
        acc = acc + jnp.dot(hs[1], wf_ref[128:256],
                            preferred_element_type=jnp.float32)
        o_ref[...] = acc + bf_ref[...]
    else:
        o_ref[:, 0:128] = hs[0]
        o_ref[:, 128:256] = hs[1]


def _post34(agg, r, r_spec, b, g, be, r_slabbed, final, wf_pad, bf_pad):
    grid = (NP // RB,)
    ocols = 128 if final else HIDDEN
    return pl.pallas_call(
        functools.partial(_post34_body, r_slabbed=r_slabbed, final=final),
        grid=grid,
        in_specs=[pl.BlockSpec((8, RB, 128), lambda i: (0, i, 0)),
                  r_spec,
                  pl.BlockSpec((2, 1, 128), lambda i: (0, 0, 0)),
                  pl.BlockSpec((2, 1, 128), lambda i: (0, 0, 0)),
                  pl.BlockSpec((2, 1, 128), lambda i: (0, 0, 0)),
                  pl.BlockSpec((HIDDEN, 128), lambda i: (0, 0)),
                  pl.BlockSpec((1, 128), lambda i: (0, 0))],
        out_specs=pl.BlockSpec((RB, ocols), lambda i: (i, 0)),
        out_shape=jax.ShapeDtypeStruct((NP, ocols), jnp.float32),
    )(agg, r, b.reshape(2, 1, 128), g.reshape(2, 1, 128),
      be.reshape(2, 1, 128), wf_pad, bf_pad)


NT = 16
ECH = 128
ROWS_PT = None


def _sc_mesh():
    return plsc.VectorSubcoreMesh(core_axis_name="c", subcore_axis_name="s")


def _sc_params():
    return pltpu.CompilerParams(needs_layout_passes=False)


def _i16():
    return lax.iota(jnp.int32, 16)


def _p1_body(tabs, esrc, edst, ew, emaskf, coefT, partH, finH,
             t0, t1, t2, t3, den0, den1, exb0, exb1, sidx_v, didx_v, mk_v,
             ewv_v, cf0_v, cf1_v, rbuf_v, acc_v):
    ept = EP // NT
    nch = ept // ECH
    rpt = NP // NT
    c = lax.axis_index("c")
    s = lax.axis_index("s")
    lane = _i16()

    pltpu.sync_copy(tabs.at[c].at[0], t0)
    pltpu.sync_copy(tabs.at[c].at[1], t1)
    pltpu.sync_copy(tabs.at[c].at[2], t2)
    pltpu.sync_copy(tabs.at[c].at[3], t3)

    def _zd(i, _):
        den0[pl.ds(i * 16, 16)] = jnp.zeros((16,), jnp.float32)
        den1[pl.ds(i * 16, 16)] = jnp.zeros((16,), jnp.float32)
        return 0
    lax.fori_loop(0, NP // 16, _zd, 0)

    def _chunk_b(ch, _):
        off = s * ept + ch * ECH
        pltpu.sync_copy(esrc.at[pl.ds(off, ECH)], sidx_v)
        pltpu.sync_copy(edst.at[pl.ds(off, ECH)], didx_v)
        pltpu.sync_copy(emaskf.at[pl.ds(off, ECH)], mk_v)
        for g in range(8):
            si = sidx_v[pl.ds(g * 16, 16)]
            di = didx_v[pl.ds(g * 16, 16)]
            mk = mk_v[pl.ds(g * 16, 16)]
            for hl, (ta, td, exb, den) in enumerate(((t0, t2, exb0, den0),
                                                     (t1, t3, exb1, den1))):
                e = plsc.load_gather(ta, [si]) + plsc.load_gather(td, [di])
                e = jnp.where(e > 0.0, e, 0.2 * e)
                ex = jnp.exp(e) * mk
                exb[pl.ds(ch * ECH + g * 16, 16)] = ex
                for l in range(16):
                    plsc.addupdate_scatter(den, [di], ex, mask=lane == l)
        return 0
    lax.fori_loop(0, nch, _chunk_b, 0)

    pltpu.sync_copy(den0, partH.at[pl.ds(((c * 2 + 0) * NT + s) * NP, NP)])
    pltpu.sync_copy(den1, partH.at[pl.ds(((c * 2 + 1) * NT + s) * NP, NP)])
    plsc.subcore_barrier()
    for hl, den in ((0, den0), (1, den1)):
        def _za(i, _):
            acc_v[pl.ds(i * 16, 16)] = jnp.zeros((16,), jnp.float32)
            return 0
        lax.fori_loop(0, rpt // 16, _za, 0)
        for p in range(NT):
            pltpu.sync_copy(
                partH.at[pl.ds(((c * 2 + hl) * NT + p) * NP + s * rpt, rpt)],
                rbuf_v)

            def _acc(g, _):
                acc_v[pl.ds(g * 16, 16)] = (acc_v[pl.ds(g * 16, 16)]
                                            + rbuf_v[pl.ds(g * 16, 16)])
                return 0
            lax.fori_loop(0, rpt // 16, _acc, 0)
        pltpu.sync_copy(acc_v,
                        finH.at[pl.ds((c * 2 + hl) * NP + s * rpt, rpt)])
    plsc.subcore_barrier()
    pltpu.sync_copy(finH.at[pl.ds((c * 2 + 0) * NP, NP)], den0)
    pltpu.sync_copy(finH.at[pl.ds((c * 2 + 1) * NP, NP)], den1)

    def _chunk_c(ch, _):
        off = s * ept + ch * ECH
        pltpu.sync_copy(edst.at[pl.ds(off, ECH)], didx_v)
        pltpu.sync_copy(ew.at[pl.ds(off, ECH)], ewv_v)
        for g in range(8):
            ewg = ewv_v[pl.ds(g * 16, 16)]
            di = didx_v[pl.ds(g * 16, 16)]
            for hl, (exb, den, cfb) in enumerate(((exb0, den0, cf0_v),
                                                  (exb1, den1, cf1_v))):
                dv = plsc.load_gather(den, [di])
                ex = exb[pl.ds(ch * ECH + g * 16, 16)]
                cfb[pl.ds(g * 16, 16)] = ex * ewg / (dv + 1e-16)
        hg = 2 * c
        pltpu.sync_copy(cf0_v, coefT.at[pl.ds(hg * EP + off, ECH)])
        pltpu.sync_copy(cf1_v, coefT.at[pl.ds((hg + 1) * EP + off, ECH)])
        return 0
    lax.fori_loop(0, nch, _chunk_c, 0)


def _p1(tabs, esrc, edst, ew, emaskf):
    ept = EP // NT
    rpt = NP // NT
    f32 = jnp.float32
    i32 = jnp.int32
    fn = pl.kernel(
        _p1_body,
        out_type=(jax.ShapeDtypeStruct((4 * EP,), f32),
                  jax.ShapeDtypeStruct((2 * 2 * NT * NP,), f32),
                  jax.ShapeDtypeStruct((2 * 2 * NP,), f32)),
        mesh=_sc_mesh(),
        compiler_params=_sc_params(),
        scratch_types=[
            pltpu.VMEM((NP,), f32), pltpu.VMEM((NP,), f32),
            pltpu.VMEM((NP,), f32), pltpu.VMEM((NP,), f32),
            pltpu.VMEM((NP,), f32), pltpu.VMEM((NP,), f32),
            pltpu.VMEM((ept,), f32), pltpu.VMEM((ept,), f32),
            pltpu.VMEM((ECH,), i32), pltpu.VMEM((ECH,), i32),
            pltpu.VMEM((ECH,), f32), pltpu.VMEM((ECH,), f32),
            pltpu.VMEM((ECH,), f32), pltpu.VMEM((ECH,), f32),
            pltpu.VMEM((rpt,), f32), pltpu.VMEM((rpt,), f32),
        ],
    )
    coefT, _, _ = fn(tabs, esrc, edst, ew, emaskf)
    return coefT


def _p3_body(hmflat, esrc, edst, coefT, aggflat,
             rows_v, sidx_v, didx_v, cb_v,
             rowsb_v, sidxb_v, didxb_v, cbb_v, outS, sga, sgb, ssa, ssb):
    ept = EP // NT
    nch = ept // ECH
    rpt = NP // NT
    c = lax.axis_index("c")
    s = lax.axis_index("s")

    for s_local in range(4):
        slab = 4 * c + s_local
        head = 2 * c + (s_local // 2)
        def _zb(i, _):
            for q in range(8):
                rows_v[i, pl.ds(q * 16, 16)] = jnp.zeros((16,), jnp.float32)
            return 0
        lax.fori_loop(0, ECH, _zb, 0)
        for j in range(rpt // ECH):
            pltpu.sync_copy(rows_v, outS.at[pl.ds(s * rpt + j * ECH, ECH)])
        plsc.subcore_barrier()

        def _load_idx(ch, sidx_v, didx_v, cb_v):
            off = s * ept + ch * ECH
            pltpu.sync_copy(esrc.at[pl.ds(off, ECH)], sidx_v)
            for g in range(8):
                sidx_v[pl.ds(g * 16, 16)] = (sidx_v[pl.ds(g * 16, 16)]
                                             + slab * NP)
            pltpu.sync_copy(edst.at[pl.ds(off, ECH)], didx_v)
            pltpu.sync_copy(coefT.at[pl.ds(head * EP + off, ECH)], cb_v)

        def _scale_all(rows_v, cb_v):
            @plsc.parallel_loop(0, ECH, 1, unroll=8)
            def _scale(r):
                spl = plsc.load_gather(cb_v, [jnp.full((16,), r, jnp.int32)])
                for q in range(8):
                    rows_v[r, pl.ds(q * 16, 16)] = (
                        rows_v[r, pl.ds(q * 16, 16)] * spl)

        _load_idx(0, sidx_v, didx_v, cb_v)
        pltpu.async_copy(hmflat.at[sidx_v], rows_v, sga)

        def _pair(i, _):
            @pl.when(i > 0)
            def _():
                pltpu.make_async_copy(hmflat.at[pl.ds(0, ECH)],
                                      outS.at[pl.ds(0, ECH)], ssb).wait()
            _load_idx(2 * i + 1, sidxb_v, didxb_v, cbb_v)
            pltpu.async_copy(hmflat.at[sidxb_v], rowsb_v, sgb)
            pltpu.make_async_copy(hmflat.at[pl.ds(0, ECH)], rows_v, sga).wait()
            _scale_all(rows_v, cb_v)
            pltpu.async_copy(rows_v, outS.at[didx_v], ssa, add=True)
            pltpu.make_async_copy(hmflat.at[pl.ds(0, ECH)],
                                  rowsb_v, sgb).wait()
            _scale_all(rowsb_v, cbb_v)
            pltpu.make_async_copy(hmflat.at[pl.ds(0, ECH)],
                                  outS.at[pl.ds(0, ECH)], ssa).wait()
            nxt = jnp.minimum(2 * i + 2, nch - 1)
            _load_idx(nxt, sidx_v, didx_v, cb_v)
            pltpu.async_copy(hmflat.at[sidx_v], rows_v, sga)
            pltpu.async_copy(rowsb_v, outS.at[didxb_v], ssb, add=True)
            return 0
        lax.fori_loop(0, nch // 2, _pair, 0)
        pltpu.make_async_copy(hmflat.at[pl.ds(0, ECH)], rows_v, sga).wait()
        pltpu.make_async_copy(hmflat.at[pl.ds(0, ECH)],
                              outS.at[pl.ds(0, ECH)], ssb).wait()
        plsc.subcore_barrier()

        for j in range(rpt // ECH):
            pltpu.sync_copy(
                outS.at[pl.ds(s * rpt + j * ECH, ECH)],
                aggflat.at[pl.ds(slab * NP + s * rpt + j * ECH, ECH)])
        plsc.subcore_barrier()


def _p3(hmflat, esrc, edst, coefT):
    f32 = jnp.float32
    i32 = jnp.int32
    fn = pl.kernel(
        _p3_body,
        out_type=jax.ShapeDtypeStruct((8 * NP, 128), f32),
        mesh=_sc_mesh(),
        compiler_params=_sc_params(),
        scratch_types=[
            pltpu.VMEM((ECH, 128), f32),
            pltpu.VMEM((ECH,), i32), pltpu.VMEM((ECH,), i32),
            pltpu.VMEM((ECH,), f32),
            pltpu.VMEM((ECH, 128), f32), pltpu.VMEM((ECH,), i32),
            pltpu.VMEM((ECH,), i32), pltpu.VMEM((ECH,), f32),
            pltpu.VMEM_SHARED((NP, 128), f32),
            pltpu.SemaphoreType.DMA, pltpu.SemaphoreType.DMA,
            pltpu.SemaphoreType.DMA, pltpu.SemaphoreType.DMA,
        ],
    )
    return fn(hmflat, esrc, edst, coefT)


def _edge_phase(hm, esrc, edst, ew, emaskf, attn_panel):
    A8t = hm[attn_panel, :, 0:8].T
    tabs = jnp.stack([jnp.stack([A8t[0], A8t[1], A8t[4], A8t[5]]),
                      jnp.stack([A8t[2], A8t[3], A8t[6], A8t[7]])])
    coefT = _p1(tabs, esrc, edst, ew, emaskf)
    hmflat = hm.reshape(-1, 128)
    aggflat = _p3(hmflat, esrc, edst, coefT)
    return aggflat.reshape(8, NP, 128)


def _build_attn_mat(a_s, a_d):
    A = jnp.zeros((HC, 16), jnp.float32)
    hid = jnp.arange(HC) // HIDDEN
    pos = jnp.arange(HC) % HIDDEN
    As = a_s[hid, pos]
    Ad = a_d[hid, pos]
    onehot = (jnp.arange(16)[None, :] == hid[:, None]).astype(jnp.float32)
    onehot4 = (jnp.arange(16)[None, :] == (hid[:, None] + 4)).astype(jnp.float32)
    A = onehot * As[:, None] + onehot4 * Ad[:, None]
    return A


def _wcat(W, a_s, a_d, Wr=None):
    A = _build_attn_mat(a_s, a_d)
    Apad = jnp.pad(W @ A, ((0, 0), (0, 112)))
    parts = [W]
    if Wr is not None:
        parts.append(Wr)
    parts.append(Apad)
    return jnp.concatenate(parts, axis=1)


def kernel(x, bn_g, bn_b, bn_mean, bn_var, W1, as1, ad1, b1, W2, as2, ad2, b2,
           W3, as3, ad3, b3, W4, as4, ad4, b4, Wr1, br1, Wr2, br2, g1, be1,
           g2, be2, g3, be3, g4, be4, Wf, bf):
    xpad = jnp.pad(x, ((0, NP - N), (0, 0)))
    xb, xn = _prep(xpad, bn_g, bn_b, bn_mean, bn_var)
    nbr, ewk, hn = _simtopk(xn)
    nbr = nbr[:N]
    ewk = ewk[:N]
    missing = hn[:N - 1, 0] == 0

    base = jnp.arange(N - 1, dtype=jnp.int32)
    esrc = jnp.concatenate([
        jnp.repeat(jnp.arange(N, dtype=jnp.int32), K), base, base + 1,
        jnp.zeros((EP - N * K - 2 * (N - 1),), jnp.int32)])
    edst = jnp.concatenate([
        nbr.reshape(-1), base + 1, base,
        jnp.zeros((EP - N * K - 2 * (N - 1),), jnp.int32)])
    ew = jnp.concatenate([
        ewk.reshape(-1), jnp.full((2 * (N - 1),), TW, jnp.float32),
        jnp.zeros((EP - N * K - 2 * (N - 1),), jnp.float32)])
    missf = missing.astype(jnp.float32)
    emaskf = jnp.concatenate([
        jnp.ones((N * K,), jnp.float32), missf, missf,
        jnp.zeros((EP - N * K - 2 * (N - 1),), jnp.float32)])

    zero8 = jnp.zeros((8 * 128,), jnp.float32)
    zero128 = jnp.zeros((128,), jnp.float32)

    Wc = _wcat(W1, as1, ad1, Wr1)
    bias = jnp.concatenate([zero8, br1, zero128])
    hm = _mm(xb, Wc, bias)
    agg = _edge_phase(hm, esrc, edst, ew, emaskf, 16)
    rspec = pl.BlockSpec((8, RB, 128), lambda i: (1, i, 0))
    h = _post12(agg, hm, rspec, b1, g1, be1, r_slabbed=True)

    Wc = _wcat(W2, as2, ad2)
    bias = jnp.concatenate([zero8, zero128])
    hm = _mm(h, Wc, bias)
    agg = _edge_phase(hm, esrc, edst, ew, emaskf, 8)
    rspec = pl.BlockSpec((RB, HC), lambda i: (i, 0))
    h = _post12(agg, h, rspec, b2, g2, be2, r_slabbed=False)

    Wc = _wcat(W3, as3, ad3, Wr2)
    bias = jnp.concatenate([zero8, br2, zero128])
    hm = _mm(h, Wc, bias)
    agg = _edge_phase(hm, esrc, edst, ew, emaskf, 10)
    rspec = pl.BlockSpec((2, RB, 128), lambda i: (4, i, 0))
    wf_pad = jnp.pad(Wf, ((0, 0), (0, 128 - NUM_CLASSES)))
    bf_pad = jnp.pad(bf, (0, 128 - NUM_CLASSES))[None]
    h = _post34(agg, hm, rspec, b3, g3, be3, True, False, wf_pad, bf_pad)

    Wc = _wcat(W4, as4, ad4)
    bias = jnp.concatenate([zero8, zero128])
    hm = _mm(h, Wc, bias)
    agg = _edge_phase(hm, esrc, edst, ew, emaskf, 8)
    rspec = pl.BlockSpec((RB, HIDDEN), lambda i: (i, 0))
    out = _post34(agg, h, rspec, b4, g4, be4, False, True, wf_pad, bf_pad)

    return out[:N, :NUM_CLASSES]


# speedup vs baseline: 12.1224x; 1.0055x over previous
import functools

import jax
import jax.numpy as jnp
from jax import lax
from jax.experimental import pallas as pl
from jax.experimental.pallas import tpu as pltpu
from jax.experimental.pallas import tpu_sc as plsc

N = 10000
EMBED = 128
HIDDEN = 256
HEADS = 4
HC = HIDDEN * HEADS
NUM_CLASSES = 64
K = 8
TW = 1.0

NP = 10240
RB = 256
CB = 512
EP = 102400
NEG = -1e30


def _prep_body(x_ref, g_ref, b_ref, m_ref, v_ref, xb_ref, xn_ref):
    x = x_ref[...]
    xb = (x - m_ref[...]) / jnp.sqrt(v_ref[...] + 1e-5) * g_ref[...] + b_ref[...]
    nrm = jnp.sqrt((xb * xb).sum(axis=1, keepdims=True))
    xb_ref[...] = xb
    xn_ref[...] = xb / (nrm + 1e-8)


def _prep(x, bn_g, bn_b, bn_mean, bn_var):
    grid = (NP // RB,)
    return pl.pallas_call(
        _prep_body,
        grid=grid,
        in_specs=[pl.BlockSpec((RB, EMBED), lambda i: (i, 0))] +
                 [pl.BlockSpec((1, EMBED), lambda i: (0, 0))] * 4,
        out_specs=[pl.BlockSpec((RB, EMBED), lambda i: (i, 0))] * 2,
        out_shape=[jax.ShapeDtypeStruct((NP, EMBED), jnp.float32)] * 2,
    )(x, bn_g[None], bn_b[None], bn_mean[None], bn_var[None])


def _simtopk_body(xr_ref, xc_ref, nbr_ref, ew_ref, hn_ref, cv_ref, cg_ref):
    i = pl.program_id(0)
    j = pl.program_id(1)
    W = 128 + CB

    @pl.when(j == 0)
    def _init():
        cv_ref[...] = jnp.full((RB, 128), NEG, jnp.float32)
        cg_ref[...] = jnp.zeros((RB, 128), jnp.float32)

    s = lax.dot_general(xr_ref[...], xc_ref[...], (((1,), (1,)), ((), ())),
                        preferred_element_type=jnp.float32)
    rowid = (jnp.float32(i * RB) +
             lax.broadcasted_iota(jnp.int32, (RB, CB), 0).astype(jnp.float32))
    colid = (jnp.float32(j * CB) +
             lax.broadcasted_iota(jnp.int32, (RB, CB), 1).astype(jnp.float32))
    s = jnp.where((colid == rowid) | (colid >= jnp.float32(N)), NEG, s)

    lanes = lax.broadcasted_iota(jnp.int32, (RB, W), 1).astype(jnp.float32)
    vb = jnp.concatenate([cv_ref[...], s], axis=1)
    gb = jnp.concatenate([cg_ref[...], colid], axis=1)
    ms, gs = [], []
    for _ in range(K):
        m = jnp.max(vb, axis=1)[:, None]
        keyloc = jnp.where(vb == m, lanes, jnp.float32(1e9))
        ni = jnp.min(keyloc, axis=1)[:, None]
        hit = keyloc == ni
        gs.append(jnp.sum(jnp.where(hit, gb, 0.0), axis=1))
        ms.append(m[:, 0])
        vb = jnp.where(hit, NEG, vb)
    lane8 = lax.broadcasted_iota(jnp.int32, (RB, 128), 1).astype(jnp.float32)
    cv = jnp.full((RB, 128), NEG, jnp.float32)
    cg = jnp.zeros((RB, 128), jnp.float32)
    for t in range(K):
        sel = lane8 == jnp.float32(t)
        cv = jnp.where(sel, ms[t][:, None], cv)
        cg = jnp.where(sel, gs[t][:, None], cg)
    cv_ref[...] = cv
    cg_ref[...] = cg

    @pl.when(j == (NP // CB) - 1)
    def _fin():
        nbf = cg_ref[:, 0:K]
        vv = cv_ref[:, 0:K]
        rid = (jnp.float32(i * RB)
               + lax.broadcasted_iota(jnp.int32, (RB, K), 0)
               .astype(jnp.float32))
        nbr_ref[...] = nbf.astype(jnp.int32)
        ew_ref[...] = vv + TW * (jnp.abs(nbf - rid) == 1).astype(jnp.float32)
        hn = (nbf == rid + 1).any(axis=1, keepdims=True)
        hn_ref[...] = jnp.broadcast_to(hn, (RB, K)).astype(jnp.int32)


def _simtopk(xn):
    grid = (NP // RB, NP // CB)
    return pl.pallas_call(
        _simtopk_body,
        grid=grid,
        in_specs=[pl.BlockSpec((RB, EMBED), lambda i, j: (i, 0)),
                  pl.BlockSpec((CB, EMBED), lambda i, j: (j, 0))],
        out_specs=[pl.BlockSpec((RB, K), lambda i, j: (i, 0))] * 3,
        out_shape=[jax.ShapeDtypeStruct((NP, K), jnp.int32),
                   jax.ShapeDtypeStruct((NP, K), jnp.float32),
                   jax.ShapeDtypeStruct((NP, K), jnp.int32)],
        scratch_shapes=[pltpu.VMEM((RB, 128), jnp.float32),
                        pltpu.VMEM((RB, 128), jnp.float32)],
    )(xn, xn)


def _mm_body(x_ref, w_ref, b_ref, o_ref):
    o_ref[0] = jnp.dot(x_ref[...], w_ref[...],
                       preferred_element_type=jnp.float32) + b_ref[0]


def _mm(x, Wcat, bias):
    Pn = Wcat.shape[1] // 128
    Kd = x.shape[1]
    grid = (NP // RB, Pn)
    return pl.pallas_call(
        _mm_body,
        grid=grid,
        in_specs=[pl.BlockSpec((RB, Kd), lambda i, j: (i, 0)),
                  pl.BlockSpec((Kd, 128), lambda i, j: (0, j)),
                  pl.BlockSpec((1, 1, 128), lambda i, j: (j, 0, 0))],
        out_specs=pl.BlockSpec((1, RB, 128), lambda i, j: (j, i, 0)),
        out_shape=jax.ShapeDtypeStruct((Pn, NP, 128), jnp.float32),
    )(x, Wcat, bias.reshape(Pn, 1, 128))


def _post12_body(agg_ref, r_ref, b_ref, g_ref, be_ref, o_ref, *, r_slabbed):
    vs = []
    ssum = 0.0
    ssq = 0.0
    for q in range(8):
        r = r_ref[q] if r_slabbed else r_ref[:, q * 128:(q + 1) * 128]
        v = jnp.maximum(agg_ref[q] + b_ref[q], 0.0) + r
        vs.append(v)
        ssum = ssum + v.sum(axis=1, keepdims=True)
        ssq = ssq + (v * v).sum(axis=1, keepdims=True)
    mu = ssum / HC
    var = ssq / HC - mu * mu
    rstd = lax.rsqrt(var + 1e-5)
    for q in range(8):
        o_ref[:, q * 128:(q + 1) * 128] = (vs[q] - mu) * rstd * g_ref[q] + be_ref[q]


def _post12(agg, r, r_spec, b, g, be, r_slabbed):
    grid = (NP // RB,)
    return pl.pallas_call(
        functools.partial(_post12_body, r_slabbed=r_slabbed),
        grid=grid,
        in_specs=[pl.BlockSpec((8, RB, 128), lambda i: (0, i, 0)),
                  r_spec,
                  pl.BlockSpec((8, 1, 128), lambda i: (0, 0, 0)),
                  pl.BlockSpec((8, 1, 128), lambda i: (0, 0, 0)),
                  pl.BlockSpec((8, 1, 128), lambda i: (0, 0, 0))],
        out_specs=pl.BlockSpec((RB, HC), lambda i: (i, 0)),
        out_shape=jax.ShapeDtypeStruct((NP, HC), jnp.float32),
    )(agg, r, b.reshape(8, 1, 128), g.reshape(8, 1, 128), be.reshape(8, 1, 128))


def _post34_body(agg_ref, r_ref, b_ref, g_ref, be_ref, wf_ref, bf_ref, o_ref,
                 *, r_slabbed, final):
    vs = []
    ssum = 0.0
    ssq = 0.0
    for p in range(2):
        m = 0.25 * (agg_ref[p] + agg_ref[2 + p] + agg_ref[4 + p] + agg_ref[6 + p])
        r = r_ref[p] if r_slabbed else r_ref[:, p * 128:(p + 1) * 128]
        v = jnp.maximum(m + b_ref[p], 0.0) + r
        vs.append(v)
        ssum = ssum + v.sum(axis=1, keepdims=True)
        ssq = ssq + (v * v).sum(axis=1, keepdims=True)
    mu = ssum / HIDDEN
    var = ssq / HIDDEN - mu * mu
    rstd = lax.rsqrt(var + 1e-5)
    hs = [(vs[p] - mu) * rstd * g_ref[p] + be_ref[p] for p in range(2)]
    if final:
        acc = jnp.dot(hs[0], wf_ref[0:128], preferred_element_type=jnp.float32)
        acc = acc + jnp.dot(hs[1], wf_ref[128:256],
                            preferred_element_type=jnp.float32)
        o_ref[...] = acc + bf_ref[...]
    else:
        o_ref[:, 0:128] = hs[0]
        o_ref[:, 128:256] = hs[1]


def _post34(agg, r, r_spec, b, g, be, r_slabbed, final, wf_pad, bf_pad):
    grid = (NP // RB,)
    ocols = 128 if final else HIDDEN
    return pl.pallas_call(
        functools.partial(_post34_body, r_slabbed=r_slabbed, final=final),
        grid=grid,
        in_specs=[pl.BlockSpec((8, RB, 128), lambda i: (0, i, 0)),
                  r_spec,
                  pl.BlockSpec((2, 1, 128), lambda i: (0, 0, 0)),
                  pl.BlockSpec((2, 1, 128), lambda i: (0, 0, 0)),
                  pl.BlockSpec((2, 1, 128), lambda i: (0, 0, 0)),
                  pl.BlockSpec((HIDDEN, 128), lambda i: (0, 0)),
                  pl.BlockSpec((1, 128), lambda i: (0, 0))],
        out_specs=pl.BlockSpec((RB, ocols), lambda i: (i, 0)),
        out_shape=jax.ShapeDtypeStruct((NP, ocols), jnp.float32),
    )(agg, r, b.reshape(2, 1, 128), g.reshape(2, 1, 128),
      be.reshape(2, 1, 128), wf_pad, bf_pad)


NT = 16
ECH = 128
ROWS_PT = None


def _sc_mesh():
    return plsc.VectorSubcoreMesh(core_axis_name="c", subcore_axis_name="s")


def _sc_params():
    return pltpu.CompilerParams(needs_layout_passes=False)


def _i16():
    return lax.iota(jnp.int32, 16)


def _p1_body(tabs, esrc, edst, ew, emaskf, coefT, partH, finH,
             t0, t1, t2, t3, den0, den1, exb0, exb1, sidx_v, didx_v, mk_v,
             ewv_v, cf0_v, cf1_v, rbuf_v, acc_v):
    ept = EP // NT
    nch = ept // ECH
    rpt = NP // NT
    c = lax.axis_index("c")
    s = lax.axis_index("s")
    lane = _i16()

    pltpu.sync_copy(tabs.at[c].at[0], t0)
    pltpu.sync_copy(tabs.at[c].at[1], t1)
    pltpu.sync_copy(tabs.at[c].at[2], t2)
    pltpu.sync_copy(tabs.at[c].at[3], t3)

    def _zd(i, _):
        den0[pl.ds(i * 16, 16)] = jnp.zeros((16,), jnp.float32)
        den1[pl.ds(i * 16, 16)] = jnp.zeros((16,), jnp.float32)
        return 0
    lax.fori_loop(0, NP // 16, _zd, 0)

    def _chunk_b(ch, _):
        off = s * ept + ch * ECH
        pltpu.sync_copy(esrc.at[pl.ds(off, ECH)], sidx_v)
        pltpu.sync_copy(edst.at[pl.ds(off, ECH)], didx_v)
        pltpu.sync_copy(emaskf.at[pl.ds(off, ECH)], mk_v)
        for g in range(8):
            si = sidx_v[pl.ds(g * 16, 16)]
            di = didx_v[pl.ds(g * 16, 16)]
            mk = mk_v[pl.ds(g * 16, 16)]
            for hl, (ta, td, exb, den) in enumerate(((t0, t2, exb0, den0),
                                                     (t1, t3, exb1, den1))):
                e = plsc.load_gather(ta, [si]) + plsc.load_gather(td, [di])
                e = jnp.where(e > 0.0, e, 0.2 * e)
                ex = jnp.exp(e) * mk
                exb[pl.ds(ch * ECH + g * 16, 16)] = ex
                for l in range(16):
                    plsc.addupdate_scatter(den, [di], ex, mask=lane == l)
        return 0
    lax.fori_loop(0, nch, _chunk_b, 0)

    pltpu.sync_copy(den0, partH.at[pl.ds(((c * 2 + 0) * NT + s) * NP, NP)])
    pltpu.sync_copy(den1, partH.at[pl.ds(((c * 2 + 1) * NT + s) * NP, NP)])
    plsc.subcore_barrier()
    for hl, den in ((0, den0), (1, den1)):
        def _za(i, _):
            acc_v[pl.ds(i * 16, 16)] = jnp.zeros((16,), jnp.float32)
            return 0
        lax.fori_loop(0, rpt // 16, _za, 0)
        for p in range(NT):
            pltpu.sync_copy(
                partH.at[pl.ds(((c * 2 + hl) * NT + p) * NP + s * rpt, rpt)],
                rbuf_v)

            def _acc(g, _):
                acc_v[pl.ds(g * 16, 16)] = (acc_v[pl.ds(g * 16, 16)]
                                            + rbuf_v[pl.ds(g * 16, 16)])
                return 0
            lax.fori_loop(0, rpt // 16, _acc, 0)
        pltpu.sync_copy(acc_v,
                        finH.at[pl.ds((c * 2 + hl) * NP + s * rpt, rpt)])
    plsc.subcore_barrier()
    pltpu.sync_copy(finH.at[pl.ds((c * 2 + 0) * NP, NP)], den0)
    pltpu.sync_copy(finH.at[pl.ds((c * 2 + 1) * NP, NP)], den1)

    def _chunk_c(ch, _):
        off = s * ept + ch * ECH
        pltpu.sync_copy(edst.at[pl.ds(off, ECH)], didx_v)
        pltpu.sync_copy(ew.at[pl.ds(off, ECH)], ewv_v)
        for g in range(8):
            ewg = ewv_v[pl.ds(g * 16, 16)]
            di = didx_v[pl.ds(g * 16, 16)]
            for hl, (exb, den, cfb) in enumerate(((exb0, den0, cf0_v),
                                                  (exb1, den1, cf1_v))):
                dv = plsc.load_gather(den, [di])
                ex = exb[pl.ds(ch * ECH + g * 16, 16)]
                cfb[pl.ds(g * 16, 16)] = ex * ewg / (dv + 1e-16)
        hg = 2 * c
        pltpu.sync_copy(cf0_v, coefT.at[pl.ds(hg * EP + off, ECH)])
        pltpu.sync_copy(cf1_v, coefT.at[pl.ds((hg + 1) * EP + off, ECH)])
        return 0
    lax.fori_loop(0, nch, _chunk_c, 0)


def _p1(tabs, esrc, edst, ew, emaskf):
    ept = EP // NT
    rpt = NP // NT
    f32 = jnp.float32
    i32 = jnp.int32
    fn = pl.kernel(
        _p1_body,
        out_type=(jax.ShapeDtypeStruct((4 * EP,), f32),
                  jax.ShapeDtypeStruct((2 * 2 * NT * NP,), f32),
                  jax.ShapeDtypeStruct((2 * 2 * NP,), f32)),
        mesh=_sc_mesh(),
        compiler_params=_sc_params(),
        scratch_types=[
            pltpu.VMEM((NP,), f32), pltpu.VMEM((NP,), f32),
            pltpu.VMEM((NP,), f32), pltpu.VMEM((NP,), f32),
            pltpu.VMEM((NP,), f32), pltpu.VMEM((NP,), f32),
            pltpu.VMEM((ept,), f32), pltpu.VMEM((ept,), f32),
            pltpu.VMEM((ECH,), i32), pltpu.VMEM((ECH,), i32),
            pltpu.VMEM((ECH,), f32), pltpu.VMEM((ECH,), f32),
            pltpu.VMEM((ECH,), f32), pltpu.VMEM((ECH,), f32),
            pltpu.VMEM((rpt,), f32), pltpu.VMEM((rpt,), f32),
        ],
    )
    coefT, _, _ = fn(tabs, esrc, edst, ew, emaskf)
    return coefT


def _p3_body(hmflat, esrc, edst, coefT, aggflat,
             rows_v, sidx_v, didx_v, cb_v,
             rowsb_v, sidxb_v, didxb_v, cbb_v, rows16_v, rows16b_v,
             outS, sga, sgb, ssa, ssb):
    ept = EP // NT
    nch = ept // ECH
    rpt = NP // NT
    c = lax.axis_index("c")
    s = lax.axis_index("s")

    def _slab_iter(s_local, _):
        slab = 4 * c + s_local
        head = 2 * c + (s_local // 2)
        def _zb(i, _):
            for q in range(8):
                rows_v[i, pl.ds(q * 16, 16)] = jnp.zeros((16,), jnp.float32)
            return 0
        lax.fori_loop(0, ECH, _zb, 0)
        for j in range(rpt // ECH):
            pltpu.sync_copy(rows_v, outS.at[pl.ds(s * rpt + j * ECH, ECH)])
        plsc.subcore_barrier()

        EK = N * K

        def _start_gather(ch, sidx_v, didx_v, cb_v, rows16_v, rows_v, sg):
            off = s * ept + ch * ECH
            pltpu.sync_copy(edst.at[pl.ds(off, ECH)], didx_v)
            pltpu.sync_copy(coefT.at[pl.ds(head * EP + off, ECH)], cb_v)

            @pl.when(off < EK)
            def _():
                base16 = pl.multiple_of(slab * NP + off // 8, 16)
                pltpu.async_copy(
                    hmflat.at[pl.ds(base16, 16)], rows16_v, sg)

            @pl.when(off >= EK)
            def _():
                pltpu.sync_copy(esrc.at[pl.ds(off, ECH)], sidx_v)
                for g in range(8):
                    sidx_v[pl.ds(g * 16, 16)] = (sidx_v[pl.ds(g * 16, 16)]
                                                 + slab * NP)
                pltpu.async_copy(hmflat.at[sidx_v], rows_v, sg)

        def _wait_scale(ch, cb_v, rows16_v, rows_v, sg):
            off = s * ept + ch * ECH

            @pl.when(off < EK)
            def _():
                pltpu.make_async_copy(hmflat.at[pl.ds(0, 16)],
                                      rows16_v, sg).wait()

                @plsc.parallel_loop(0, ECH // 8, 1, unroll=4)
                def _sk(rr):
                    for u in range(8):
                        r = rr * 8 + u
                        spl = plsc.load_gather(
                            cb_v, [jnp.full((16,), r, jnp.int32)])
                        for q in range(8):
                            rows_v[r, pl.ds(q * 16, 16)] = (
                                rows16_v[rr, pl.ds(q * 16, 16)] * spl)

            @pl.when(off >= EK)
            def _():
                pltpu.make_async_copy(hmflat.at[pl.ds(0, ECH)],
                                      rows_v, sg).wait()

                @plsc.parallel_loop(0, ECH, 1, unroll=8)
                def _sg(r):
                    spl = plsc.load_gather(
                        cb_v, [jnp.full((16,), r, jnp.int32)])
                    for q in range(8):
                        rows_v[r, pl.ds(q * 16, 16)] = (
                            rows_v[r, pl.ds(q * 16, 16)] * spl)

        _start_gather(0, sidx_v, didx_v, cb_v, rows16_v, rows_v, sga)

        def _pair(i, _):
            @pl.when(i > 0)
            def _():
                pltpu.make_async_copy(hmflat.at[pl.ds(0, ECH)],
                                      outS.at[pl.ds(0, ECH)], ssb).wait()
            _start_gather(2 * i + 1, sidxb_v, didxb_v, cbb_v,
                          rows16b_v, rowsb_v, sgb)
            _wait_scale(2 * i, cb_v, rows16_v, rows_v, sga)
            pltpu.async_copy(rows_v, outS.at[didx_v], ssa, add=True)
            _wait_scale(2 * i + 1, cbb_v, rows16b_v, rowsb_v, sgb)
            pltpu.make_async_copy(hmflat.at[pl.ds(0, ECH)],
                                  outS.at[pl.ds(0, ECH)], ssa).wait()
            nxt = jnp.minimum(2 * i + 2, nch - 1)
            _start_gather(nxt, sidx_v, didx_v, cb_v, rows16_v, rows_v, sga)
            pltpu.async_copy(rowsb_v, outS.at[didxb_v], ssb, add=True)
            return 0
        lax.fori_loop(0, nch // 2, _pair, 0)
        lastch = nch - 1
        _wait_scale(lastch, cb_v, rows16_v, rows_v, sga)
        pltpu.make_async_copy(hmflat.at[pl.ds(0, ECH)],
                              outS.at[pl.ds(0, ECH)], ssb).wait()
        plsc.subcore_barrier()

        for j in range(rpt // ECH):
            pltpu.sync_copy(
                outS.at[pl.ds(s * rpt + j * ECH, ECH)],
                aggflat.at[pl.ds(slab * NP + s * rpt + j * ECH, ECH)])
        plsc.subcore_barrier()
        return 0

    lax.fori_loop(0, 4, _slab_iter, 0)


def _p3(hmflat, esrc, edst, coefT):
    f32 = jnp.float32
    i32 = jnp.int32
    fn = pl.kernel(
        _p3_body,
        out_type=jax.ShapeDtypeStruct((8 * NP, 128), f32),
        mesh=_sc_mesh(),
        compiler_params=_sc_params(),
        scratch_types=[
            pltpu.VMEM((ECH, 128), f32),
            pltpu.VMEM((ECH,), i32), pltpu.VMEM((ECH,), i32),
            pltpu.VMEM((ECH,), f32),
            pltpu.VMEM((ECH, 128), f32), pltpu.VMEM((ECH,), i32),
            pltpu.VMEM((ECH,), i32), pltpu.VMEM((ECH,), f32),
            pltpu.VMEM((16, 128), f32), pltpu.VMEM((16, 128), f32),
            pltpu.VMEM_SHARED((NP, 128), f32),
            pltpu.SemaphoreType.DMA, pltpu.SemaphoreType.DMA,
            pltpu.SemaphoreType.DMA, pltpu.SemaphoreType.DMA,
        ],
    )
    return fn(hmflat, esrc, edst, coefT)


def _edge_phase(hm, esrc, edst, ew, emaskf, attn_panel):
    A8t = hm[attn_panel, :, 0:8].T
    tabs = jnp.stack([jnp.stack([A8t[0], A8t[1], A8t[4], A8t[5]]),
                      jnp.stack([A8t[2], A8t[3], A8t[6], A8t[7]])])
    coefT = _p1(tabs, esrc, edst, ew, emaskf)
    hmflat = hm.reshape(-1, 128)
    aggflat = _p3(hmflat, esrc, edst, coefT)
    return aggflat.reshape(8, NP, 128)


def _build_attn_mat(a_s, a_d):
    A = jnp.zeros((HC, 16), jnp.float32)
    hid = jnp.arange(HC) // HIDDEN
    pos = jnp.arange(HC) % HIDDEN
    As = a_s[hid, pos]
    Ad = a_d[hid, pos]
    onehot = (jnp.arange(16)[None, :] == hid[:, None]).astype(jnp.float32)
    onehot4 = (jnp.arange(16)[None, :] == (hid[:, None] + 4)).astype(jnp.float32)
    A = onehot * As[:, None] + onehot4 * Ad[:, None]
    return A


def _wcat(W, a_s, a_d, Wr=None):
    A = _build_attn_mat(a_s, a_d)
    Apad = jnp.pad(W @ A, ((0, 0), (0, 112)))
    parts = [W]
    if Wr is not None:
        parts.append(Wr)
    parts.append(Apad)
    return jnp.concatenate(parts, axis=1)


def kernel(x, bn_g, bn_b, bn_mean, bn_var, W1, as1, ad1, b1, W2, as2, ad2, b2,
           W3, as3, ad3, b3, W4, as4, ad4, b4, Wr1, br1, Wr2, br2, g1, be1,
           g2, be2, g3, be3, g4, be4, Wf, bf):
    xpad = jnp.pad(x, ((0, NP - N), (0, 0)))
    xb, xn = _prep(xpad, bn_g, bn_b, bn_mean, bn_var)
    nbr, ewk, hn = _simtopk(xn)
    nbr = nbr[:N]
    ewk = ewk[:N]
    missing = hn[:N - 1, 0] == 0

    base = jnp.arange(N - 1, dtype=jnp.int32)
    esrc = jnp.concatenate([
        jnp.repeat(jnp.arange(N, dtype=jnp.int32), K), base, base + 1,
        jnp.zeros((EP - N * K - 2 * (N - 1),), jnp.int32)])
    edst = jnp.concatenate([
        nbr.reshape(-1), base + 1, base,
        jnp.zeros((EP - N * K - 2 * (N - 1),), jnp.int32)])
    ew = jnp.concatenate([
        ewk.reshape(-1), jnp.full((2 * (N - 1),), TW, jnp.float32),
        jnp.zeros((EP - N * K - 2 * (N - 1),), jnp.float32)])
    missf = missing.astype(jnp.float32)
    emaskf = jnp.concatenate([
        jnp.ones((N * K,), jnp.float32), missf, missf,
        jnp.zeros((EP - N * K - 2 * (N - 1),), jnp.float32)])

    zero8 = jnp.zeros((8 * 128,), jnp.float32)
    zero128 = jnp.zeros((128,), jnp.float32)

    Wc = _wcat(W1, as1, ad1, Wr1)
    bias = jnp.concatenate([zero8, br1, zero128])
    hm = _mm(xb, Wc, bias)
    agg = _edge_phase(hm, esrc, edst, ew, emaskf, 16)
    rspec = pl.BlockSpec((8, RB, 128), lambda i: (1, i, 0))
    h = _post12(agg, hm, rspec, b1, g1, be1, r_slabbed=True)

    Wc = _wcat(W2, as2, ad2)
    bias = jnp.concatenate([zero8, zero128])
    hm = _mm(h, Wc, bias)
    agg = _edge_phase(hm, esrc, edst, ew, emaskf, 8)
    rspec = pl.BlockSpec((RB, HC), lambda i: (i, 0))
    h = _post12(agg, h, rspec, b2, g2, be2, r_slabbed=False)

    Wc = _wcat(W3, as3, ad3, Wr2)
    bias = jnp.concatenate([zero8, br2, zero128])
    hm = _mm(h, Wc, bias)
    agg = _edge_phase(hm, esrc, edst, ew, emaskf, 10)
    rspec = pl.BlockSpec((2, RB, 128), lambda i: (4, i, 0))
    wf_pad = jnp.pad(Wf, ((0, 0), (0, 128 - NUM_CLASSES)))
    bf_pad = jnp.pad(bf, (0, 128 - NUM_CLASSES))[None]
    h = _post34(agg, hm, rspec, b3, g3, be3, True, False, wf_pad, bf_pad)

    Wc = _wcat(W4, as4, ad4)
    bias = jnp.concatenate([zero8, zero128])
    hm = _mm(h, Wc, bias)
    agg = _edge_phase(hm, esrc, edst, ew, emaskf, 8)
    rspec = pl.BlockSpec((RB, HIDDEN), lambda i: (i, 0))
    out = _post34(agg, h, rspec, b4, g4, be4, False, True, wf_pad, bf_pad)

    return out[:N, :NUM_CLASSES]


# speedup vs baseline: 12.5543x; 1.0356x over previous
import functools

import jax
import jax.numpy as jnp
from jax import lax
from jax.experimental import pallas as pl
from jax.experimental.pallas import tpu as pltpu
from jax.experimental.pallas import tpu_sc as plsc

N = 10000
EMBED = 128
HIDDEN = 256
HEADS = 4
HC = HIDDEN * HEADS
NUM_CLASSES = 64
K = 8
TW = 1.0

NP = 10240
RB = 256
CB = 1024
EP = 102400
NEG = -1e30


def _prep_body(x_ref, g_ref, b_ref, m_ref, v_ref, xb_ref, xn_ref):
    x = x_ref[...]
    xb = (x - m_ref[...]) / jnp.sqrt(v_ref[...] + 1e-5) * g_ref[...] + b_ref[...]
    nrm = jnp.sqrt((xb * xb).sum(axis=1, keepdims=True))
    xb_ref[...] = xb
    xn_ref[...] = xb / (nrm + 1e-8)


def _prep(x, bn_g, bn_b, bn_mean, bn_var):
    grid = (NP // RB,)
    return pl.pallas_call(
        _prep_body,
        grid=grid,
        in_specs=[pl.BlockSpec((RB, EMBED), lambda i: (i, 0))] +
                 [pl.BlockSpec((1, EMBED), lambda i: (0, 0))] * 4,
        out_specs=[pl.BlockSpec((RB, EMBED), lambda i: (i, 0))] * 2,
        out_shape=[jax.ShapeDtypeStruct((NP, EMBED), jnp.float32)] * 2,
    )(x, bn_g[None], bn_b[None], bn_mean[None], bn_var[None])


def _simtopk_body(xr_ref, xc_ref, nbr_ref, ew_ref, hn_ref, cv_ref, cg_ref):
    i = pl.program_id(0)
    j = pl.program_id(1)
    W = 128 + CB

    @pl.when(j == 0)
    def _init():
        cv_ref[...] = jnp.full((RB, 128), NEG, jnp.float32)
        cg_ref[...] = jnp.zeros((RB, 128), jnp.float32)

    s = lax.dot_general(xr_ref[...], xc_ref[...], (((1,), (1,)), ((), ())),
                        preferred_element_type=jnp.float32)
    rowid = (jnp.float32(i * RB) +
             lax.broadcasted_iota(jnp.int32, (RB, CB), 0).astype(jnp.float32))
    colid = (jnp.float32(j * CB) +
             lax.broadcasted_iota(jnp.int32, (RB, CB), 1).astype(jnp.float32))
    s = jnp.where((colid == rowid) | (colid >= jnp.float32(N)), NEG, s)

    lanes = lax.broadcasted_iota(jnp.int32, (RB, W), 1).astype(jnp.float32)
    vb = jnp.concatenate([cv_ref[...], s], axis=1)
    gb = jnp.concatenate([cg_ref[...], colid], axis=1)
    ms, gs = [], []
    for _ in range(K):
        m = jnp.max(vb, axis=1)[:, None]
        keyloc = jnp.where(vb == m, lanes, jnp.float32(1e9))
        ni = jnp.min(keyloc, axis=1)[:, None]
        hit = keyloc == ni
        gs.append(jnp.sum(jnp.where(hit, gb, 0.0), axis=1))
        ms.append(m[:, 0])
        vb = jnp.where(hit, NEG, vb)
    lane8 = lax.broadcasted_iota(jnp.int32, (RB, 128), 1).astype(jnp.float32)
    cv = jnp.full((RB, 128), NEG, jnp.float32)
    cg = jnp.zeros((RB, 128), jnp.float32)
    for t in range(K):
        sel = lane8 == jnp.float32(t)
        cv = jnp.where(sel, ms[t][:, None], cv)
        cg = jnp.where(sel, gs[t][:, None], cg)
    cv_ref[...] = cv
    cg_ref[...] = cg

    @pl.when(j == (NP // CB) - 1)
    def _fin():
        nbf = cg_ref[:, 0:K]
        vv = cv_ref[:, 0:K]
        rid = (jnp.float32(i * RB)
               + lax.broadcasted_iota(jnp.int32, (RB, K), 0)
               .astype(jnp.float32))
        nbr_ref[...] = nbf.astype(jnp.int32)
        ew_ref[...] = vv + TW * (jnp.abs(nbf - rid) == 1).astype(jnp.float32)
        hn = (nbf == rid + 1).any(axis=1, keepdims=True)
        hn_ref[...] = jnp.broadcast_to(hn, (RB, K)).astype(jnp.int32)


def _simtopk(xn):
    grid = (NP // RB, NP // CB)
    return pl.pallas_call(
        _simtopk_body,
        grid=grid,
        in_specs=[pl.BlockSpec((RB, EMBED), lambda i, j: (i, 0)),
                  pl.BlockSpec((CB, EMBED), lambda i, j: (j, 0))],
        out_specs=[pl.BlockSpec((RB, K), lambda i, j: (i, 0))] * 3,
        out_shape=[jax.ShapeDtypeStruct((NP, K), jnp.int32),
                   jax.ShapeDtypeStruct((NP, K), jnp.float32),
                   jax.ShapeDtypeStruct((NP, K), jnp.int32)],
        scratch_shapes=[pltpu.VMEM((RB, 128), jnp.float32),
                        pltpu.VMEM((RB, 128), jnp.float32)],
    )(xn, xn)


def _mm_body(x_ref, w_ref, b_ref, o_ref):
    o_ref[0] = jnp.dot(x_ref[...], w_ref[...],
                       preferred_element_type=jnp.float32) + b_ref[0]


def _mm(x, Wcat, bias):
    Pn = Wcat.shape[1] // 128
    Kd = x.shape[1]
    grid = (NP // RB, Pn)
    return pl.pallas_call(
        _mm_body,
        grid=grid,
        in_specs=[pl.BlockSpec((RB, Kd), lambda i, j: (i, 0)),
                  pl.BlockSpec((Kd, 128), lambda i, j: (0, j)),
                  pl.BlockSpec((1, 1, 128), lambda i, j: (j, 0, 0))],
        out_specs=pl.BlockSpec((1, RB, 128), lambda i, j: (j, i, 0)),
        out_shape=jax.ShapeDtypeStruct((Pn, NP, 128), jnp.float32),
    )(x, Wcat, bias.reshape(Pn, 1, 128))


def _post12_body(agg_ref, r_ref, b_ref, g_ref, be_ref, o_ref, *, r_slabbed):
    vs = []
    ssum = 0.0
    ssq = 0.0
    for q in range(8):
        r = r_ref[q] if r_slabbed else r_ref[:, q * 128:(q + 1) * 128]
        v = jnp.maximum(agg_ref[q] + b_ref[q], 0.0) + r
        vs.append(v)
        ssum = ssum + v.sum(axis=1, keepdims=True)
        ssq = ssq + (v * v).sum(axis=1, keepdims=True)
    mu = ssum / HC
    var = ssq / HC - mu * mu
    rstd = lax.rsqrt(var + 1e-5)
    for q in range(8):
        o_ref[:, q * 128:(q + 1) * 128] = (vs[q] - mu) * rstd * g_ref[q] + be_ref[q]


def _post12(agg, r, r_spec, b, g, be, r_slabbed):
    grid = (NP // RB,)
    return pl.pallas_call(
        functools.partial(_post12_body, r_slabbed=r_slabbed),
        grid=grid,
        in_specs=[pl.BlockSpec((8, RB, 128), lambda i: (0, i, 0)),
                  r_spec,
                  pl.BlockSpec((8, 1, 128), lambda i: (0, 0, 0)),
                  pl.BlockSpec((8, 1, 128), lambda i: (0, 0, 0)),
                  pl.BlockSpec((8, 1, 128), lambda i: (0, 0, 0))],
        out_specs=pl.BlockSpec((RB, HC), lambda i: (i, 0)),
        out_shape=jax.ShapeDtypeStruct((NP, HC), jnp.float32),
    )(agg, r, b.reshape(8, 1, 128), g.reshape(8, 1, 128), be.reshape(8, 1, 128))


def _post34_body(agg_ref, r_ref, b_ref, g_ref, be_ref, wf_ref, bf_ref, o_ref,
                 *, r_slabbed, final):
    vs = []
    ssum = 0.0
    ssq = 0.0
    for p in range(2):
        m = 0.25 * (agg_ref[p] + agg_ref[2 + p] + agg_ref[4 + p] + agg_ref[6 + p])
        r = r_ref[p] if r_slabbed else r_ref[:, p * 128:(p + 1) * 128]
        v = jnp.maximum(m + b_ref[p], 0.0) + r
        vs.append(v)
        ssum = ssum + v.sum(axis=1, keepdims=True)
        ssq = ssq + (v * v).sum(axis=1, keepdims=True)
    mu = ssum / HIDDEN
    var = ssq / HIDDEN - mu * mu
    rstd = lax.rsqrt(var + 1e-5)
    hs = [(vs[p] - mu) * rstd * g_ref[p] + be_ref[p] for p in range(2)]
    if final:
        acc = jnp.dot(hs[0], wf_ref[0:128], preferred_element_type=jnp.float32)
        acc = acc + jnp.dot(hs[1], wf_ref[128:256],
                            preferred_element_type=jnp.float32)
        o_ref[...] = acc + bf_ref[...]
    else:
        o_ref[:, 0:128] = hs[0]
        o_ref[:, 128:256] = hs[1]


def _post34(agg, r, r_spec, b, g, be, r_slabbed, final, wf_pad, bf_pad):
    grid = (NP // RB,)
    ocols = 128 if final else HIDDEN
    return pl.pallas_call(
        functools.partial(_post34_body, r_slabbed=r_slabbed, final=final),
        grid=grid,
        in_specs=[pl.BlockSpec((8, RB, 128), lambda i: (0, i, 0)),
                  r_spec,
                  pl.BlockSpec((2, 1, 128), lambda i: (0, 0, 0)),
                  pl.BlockSpec((2, 1, 128), lambda i: (0, 0, 0)),
                  pl.BlockSpec((2, 1, 128), lambda i: (0, 0, 0)),
                  pl.BlockSpec((HIDDEN, 128), lambda i: (0, 0)),
                  pl.BlockSpec((1, 128), lambda i: (0, 0))],
        out_specs=pl.BlockSpec((RB, ocols), lambda i: (i, 0)),
        out_shape=jax.ShapeDtypeStruct((NP, ocols), jnp.float32),
    )(agg, r, b.reshape(2, 1, 128), g.reshape(2, 1, 128),
      be.reshape(2, 1, 128), wf_pad, bf_pad)


NT = 16
ECH = 128
ROWS_PT = None


def _sc_mesh():
    return plsc.VectorSubcoreMesh(core_axis_name="c", subcore_axis_name="s")


def _sc_params():
    return pltpu.CompilerParams(needs_layout_passes=False)


def _i16():
    return lax.iota(jnp.int32, 16)


def _p1_body(tabs, esrc, edst, ew, emaskf, coefT, partH, finH,
             t0, t1, t2, t3, den0, den1, exb0, exb1, sidx_v, didx_v, mk_v,
             ewv_v, cf0_v, cf1_v, rbuf_v, acc_v):
    ept = EP // NT
    nch = ept // ECH
    rpt = NP // NT
    c = lax.axis_index("c")
    s = lax.axis_index("s")
    lane = _i16()

    pltpu.sync_copy(tabs.at[c].at[0], t0)
    pltpu.sync_copy(tabs.at[c].at[1], t1)
    pltpu.sync_copy(tabs.at[c].at[2], t2)
    pltpu.sync_copy(tabs.at[c].at[3], t3)

    def _zd(i, _):
        den0[pl.ds(i * 16, 16)] = jnp.zeros((16,), jnp.float32)
        den1[pl.ds(i * 16, 16)] = jnp.zeros((16,), jnp.float32)
        return 0
    lax.fori_loop(0, NP // 16, _zd, 0)

    def _chunk_b(ch, _):
        off = s * ept + ch * ECH
        pltpu.sync_copy(esrc.at[pl.ds(off, ECH)], sidx_v)
        pltpu.sync_copy(edst.at[pl.ds(off, ECH)], didx_v)
        pltpu.sync_copy(emaskf.at[pl.ds(off, ECH)], mk_v)
        for g in range(8):
            si = sidx_v[pl.ds(g * 16, 16)]
            di = didx_v[pl.ds(g * 16, 16)]
            mk = mk_v[pl.ds(g * 16, 16)]
            for hl, (ta, td, exb, den) in enumerate(((t0, t2, exb0, den0),
                                                     (t1, t3, exb1, den1))):
                e = plsc.load_gather(ta, [si]) + plsc.load_gather(td, [di])
                e = jnp.where(e > 0.0, e, 0.2 * e)
                ex = jnp.exp(e) * mk
                exb[pl.ds(ch * ECH + g * 16, 16)] = ex
                for l in range(16):
                    plsc.addupdate_scatter(den, [di], ex, mask=lane == l)
        return 0
    lax.fori_loop(0, nch, _chunk_b, 0)

    pltpu.sync_copy(den0, partH.at[pl.ds(((c * 2 + 0) * NT + s) * NP, NP)])
    pltpu.sync_copy(den1, partH.at[pl.ds(((c * 2 + 1) * NT + s) * NP, NP)])
    plsc.subcore_barrier()
    for hl, den in ((0, den0), (1, den1)):
        def _za(i, _):
            acc_v[pl.ds(i * 16, 16)] = jnp.zeros((16,), jnp.float32)
            return 0
        lax.fori_loop(0, rpt // 16, _za, 0)
        for p in range(NT):
            pltpu.sync_copy(
                partH.at[pl.ds(((c * 2 + hl) * NT + p) * NP + s * rpt, rpt)],
                rbuf_v)

            def _acc(g, _):
                acc_v[pl.ds(g * 16, 16)] = (acc_v[pl.ds(g * 16, 16)]
                                            + rbuf_v[pl.ds(g * 16, 16)])
                return 0
            lax.fori_loop(0, rpt // 16, _acc, 0)
        pltpu.sync_copy(acc_v,
                        finH.at[pl.ds((c * 2 + hl) * NP + s * rpt, rpt)])
    plsc.subcore_barrier()
    pltpu.sync_copy(finH.at[pl.ds((c * 2 + 0) * NP, NP)], den0)
    pltpu.sync_copy(finH.at[pl.ds((c * 2 + 1) * NP, NP)], den1)

    def _chunk_c(ch, _):
        off = s * ept + ch * ECH
        pltpu.sync_copy(edst.at[pl.ds(off, ECH)], didx_v)
        pltpu.sync_copy(ew.at[pl.ds(off, ECH)], ewv_v)
        for g in range(8):
            ewg = ewv_v[pl.ds(g * 16, 16)]
            di = didx_v[pl.ds(g * 16, 16)]
            for hl, (exb, den, cfb) in enumerate(((exb0, den0, cf0_v),
                                                  (exb1, den1, cf1_v))):
                dv = plsc.load_gather(den, [di])
                ex = exb[pl.ds(ch * ECH + g * 16, 16)]
                cfb[pl.ds(g * 16, 16)] = ex * ewg / (dv + 1e-16)
        hg = 2 * c
        pltpu.sync_copy(cf0_v, coefT.at[pl.ds(hg * EP + off, ECH)])
        pltpu.sync_copy(cf1_v, coefT.at[pl.ds((hg + 1) * EP + off, ECH)])
        return 0
    lax.fori_loop(0, nch, _chunk_c, 0)


def _p1(tabs, esrc, edst, ew, emaskf):
    ept = EP // NT
    rpt = NP // NT
    f32 = jnp.float32
    i32 = jnp.int32
    fn = pl.kernel(
        _p1_body,
        out_type=(jax.ShapeDtypeStruct((4 * EP,), f32),
                  jax.ShapeDtypeStruct((2 * 2 * NT * NP,), f32),
                  jax.ShapeDtypeStruct((2 * 2 * NP,), f32)),
        mesh=_sc_mesh(),
        compiler_params=_sc_params(),
        scratch_types=[
            pltpu.VMEM((NP,), f32), pltpu.VMEM((NP,), f32),
            pltpu.VMEM((NP,), f32), pltpu.VMEM((NP,), f32),
            pltpu.VMEM((NP,), f32), pltpu.VMEM((NP,), f32),
            pltpu.VMEM((ept,), f32), pltpu.VMEM((ept,), f32),
            pltpu.VMEM((ECH,), i32), pltpu.VMEM((ECH,), i32),
            pltpu.VMEM((ECH,), f32), pltpu.VMEM((ECH,), f32),
            pltpu.VMEM((ECH,), f32), pltpu.VMEM((ECH,), f32),
            pltpu.VMEM((rpt,), f32), pltpu.VMEM((rpt,), f32),
        ],
    )
    coefT, _, _ = fn(tabs, esrc, edst, ew, emaskf)
    return coefT


def _p3_body(hmflat, esrc, edst, coefT, aggflat,
             rows_v, sidx_v, didx_v, cb_v,
             rowsb_v, sidxb_v, didxb_v, cbb_v, rows16_v, rows16b_v,
             outS, sga, sgb, ssa, ssb):
    ept = EP // NT
    nch = ept // ECH
    rpt = NP // NT
    c = lax.axis_index("c")
    s = lax.axis_index("s")

    def _slab_iter(s_local, _):
        slab = 4 * c + s_local
        head = 2 * c + (s_local // 2)
        def _zb(i, _):
            for q in range(8):
                rows_v[i, pl.ds(q * 16, 16)] = jnp.zeros((16,), jnp.float32)
            return 0
        lax.fori_loop(0, ECH, _zb, 0)
        for j in range(rpt // ECH):
            pltpu.sync_copy(rows_v, outS.at[pl.ds(s * rpt + j * ECH, ECH)])
        plsc.subcore_barrier()

        EK = N * K

        def _start_gather(ch, sidx_v, didx_v, cb_v, rows16_v, rows_v, sg):
            off = s * ept + ch * ECH
            pltpu.sync_copy(edst.at[pl.ds(off, ECH)], didx_v)
            pltpu.sync_copy(coefT.at[pl.ds(head * EP + off, ECH)], cb_v)

            @pl.when(off < EK)
            def _():
                base16 = pl.multiple_of(slab * NP + off // 8, 16)
                pltpu.async_copy(
                    hmflat.at[pl.ds(base16, 16)], rows16_v, sg)

            @pl.when(off >= EK)
            def _():
                pltpu.sync_copy(esrc.at[pl.ds(off, ECH)], sidx_v)
                for g in range(8):
                    sidx_v[pl.ds(g * 16, 16)] = (sidx_v[pl.ds(g * 16, 16)]
                                                 + slab * NP)
                pltpu.async_copy(hmflat.at[sidx_v], rows_v, sg)

        def _wait_scale(ch, cb_v, rows16_v, rows_v, sg):
            off = s * ept + ch * ECH

            @pl.when(off < EK)
            def _():
                pltpu.make_async_copy(hmflat.at[pl.ds(0, 16)],
                                      rows16_v, sg).wait()

                @plsc.parallel_loop(0, ECH // 8, 1, unroll=4)
                def _sk(rr):
                    for u in range(8):
                        r = rr * 8 + u
                        spl = plsc.load_gather(
                            cb_v, [jnp.full((16,), r, jnp.int32)])
                        for q in range(8):
                            rows_v[r, pl.ds(q * 16, 16)] = (
                                rows16_v[rr, pl.ds(q * 16, 16)] * spl)

            @pl.when(off >= EK)
            def _():
                pltpu.make_async_copy(hmflat.at[pl.ds(0, ECH)],
                                      rows_v, sg).wait()

                @plsc.parallel_loop(0, ECH, 1, unroll=8)
                def _sg(r):
                    spl = plsc.load_gather(
                        cb_v, [jnp.full((16,), r, jnp.int32)])
                    for q in range(8):
                        rows_v[r, pl.ds(q * 16, 16)] = (
                            rows_v[r, pl.ds(q * 16, 16)] * spl)

        _start_gather(0, sidx_v, didx_v, cb_v, rows16_v, rows_v, sga)

        def _pair(i, _):
            @pl.when(i > 0)
            def _():
                pltpu.make_async_copy(hmflat.at[pl.ds(0, ECH)],
                                      outS.at[pl.ds(0, ECH)], ssb).wait()
            _start_gather(2 * i + 1, sidxb_v, didxb_v, cbb_v,
                          rows16b_v, rowsb_v, sgb)
            _wait_scale(2 * i, cb_v, rows16_v, rows_v, sga)
            pltpu.async_copy(rows_v, outS.at[didx_v], ssa, add=True)
            _wait_scale(2 * i + 1, cbb_v, rows16b_v, rowsb_v, sgb)
            pltpu.make_async_copy(hmflat.at[pl.ds(0, ECH)],
                                  outS.at[pl.ds(0, ECH)], ssa).wait()
            nxt = jnp.minimum(2 * i + 2, nch - 1)
            _start_gather(nxt, sidx_v, didx_v, cb_v, rows16_v, rows_v, sga)
            pltpu.async_copy(rowsb_v, outS.at[didxb_v], ssb, add=True)
            return 0
        lax.fori_loop(0, nch // 2, _pair, 0)
        lastch = nch - 1
        _wait_scale(lastch, cb_v, rows16_v, rows_v, sga)
        pltpu.make_async_copy(hmflat.at[pl.ds(0, ECH)],
                              outS.at[pl.ds(0, ECH)], ssb).wait()
        plsc.subcore_barrier()

        for j in range(rpt // ECH):
            pltpu.sync_copy(
                outS.at[pl.ds(s * rpt + j * ECH, ECH)],
                aggflat.at[pl.ds(slab * NP + s * rpt + j * ECH, ECH)])
        plsc.subcore_barrier()
        return 0

    lax.fori_loop(0, 4, _slab_iter, 0)


def _p3(hmflat, esrc, edst, coefT):
    f32 = jnp.float32
    i32 = jnp.int32
    fn = pl.kernel(
        _p3_body,
        out_type=jax.ShapeDtypeStruct((8 * NP, 128), f32),
        mesh=_sc_mesh(),
        compiler_params=_sc_params(),
        scratch_types=[
            pltpu.VMEM((ECH, 128), f32),
            pltpu.VMEM((ECH,), i32), pltpu.VMEM((ECH,), i32),
            pltpu.VMEM((ECH,), f32),
            pltpu.VMEM((ECH, 128), f32), pltpu.VMEM((ECH,), i32),
            pltpu.VMEM((ECH,), i32), pltpu.VMEM((ECH,), f32),
            pltpu.VMEM((16, 128), f32), pltpu.VMEM((16, 128), f32),
            pltpu.VMEM_SHARED((NP, 128), f32),
            pltpu.SemaphoreType.DMA, pltpu.SemaphoreType.DMA,
            pltpu.SemaphoreType.DMA, pltpu.SemaphoreType.DMA,
        ],
    )
    return fn(hmflat, esrc, edst, coefT)


def _edge_phase(hm, esrc, edst, ew, emaskf, attn_panel):
    A8t = hm[attn_panel, :, 0:8].T
    tabs = jnp.stack([jnp.stack([A8t[0], A8t[1], A8t[4], A8t[5]]),
                      jnp.stack([A8t[2], A8t[3], A8t[6], A8t[7]])])
    coefT = _p1(tabs, esrc, edst, ew, emaskf)
    hmflat = hm.reshape(-1, 128)
    aggflat = _p3(hmflat, esrc, edst, coefT)
    return aggflat.reshape(8, NP, 128)


def _build_attn_mat(a_s, a_d):
    A = jnp.zeros((HC, 16), jnp.float32)
    hid = jnp.arange(HC) // HIDDEN
    pos = jnp.arange(HC) % HIDDEN
    As = a_s[hid, pos]
    Ad = a_d[hid, pos]
    onehot = (jnp.arange(16)[None, :] == hid[:, None]).astype(jnp.float32)
    onehot4 = (jnp.arange(16)[None, :] == (hid[:, None] + 4)).astype(jnp.float32)
    A = onehot * As[:, None] + onehot4 * Ad[:, None]
    return A


def _wcat(W, a_s, a_d, Wr=None):
    A = _build_attn_mat(a_s, a_d)
    Apad = jnp.pad(W @ A, ((0, 0), (0, 112)))
    parts = [W]
    if Wr is not None:
        parts.append(Wr)
    parts.append(Apad)
    return jnp.concatenate(parts, axis=1)


def kernel(x, bn_g, bn_b, bn_mean, bn_var, W1, as1, ad1, b1, W2, as2, ad2, b2,
           W3, as3, ad3, b3, W4, as4, ad4, b4, Wr1, br1, Wr2, br2, g1, be1,
           g2, be2, g3, be3, g4, be4, Wf, bf):
    xpad = jnp.pad(x, ((0, NP - N), (0, 0)))
    xb, xn = _prep(xpad, bn_g, bn_b, bn_mean, bn_var)
    nbr, ewk, hn = _simtopk(xn)
    nbr = nbr[:N]
    ewk = ewk[:N]
    missing = hn[:N - 1, 0] == 0

    base = jnp.arange(N - 1, dtype=jnp.int32)
    esrc = jnp.concatenate([
        jnp.repeat(jnp.arange(N, dtype=jnp.int32), K), base, base + 1,
        jnp.zeros((EP - N * K - 2 * (N - 1),), jnp.int32)])
    edst = jnp.concatenate([
        nbr.reshape(-1), base + 1, base,
        jnp.zeros((EP - N * K - 2 * (N - 1),), jnp.int32)])
    ew = jnp.concatenate([
        ewk.reshape(-1), jnp.full((2 * (N - 1),), TW, jnp.float32),
        jnp.zeros((EP - N * K - 2 * (N - 1),), jnp.float32)])
    missf = missing.astype(jnp.float32)
    emaskf = jnp.concatenate([
        jnp.ones((N * K,), jnp.float32), missf, missf,
        jnp.zeros((EP - N * K - 2 * (N - 1),), jnp.float32)])

    zero8 = jnp.zeros((8 * 128,), jnp.float32)
    zero128 = jnp.zeros((128,), jnp.float32)

    Wc = _wcat(W1, as1, ad1, Wr1)
    bias = jnp.concatenate([zero8, br1, zero128])
    hm = _mm(xb, Wc, bias)
    agg = _edge_phase(hm, esrc, edst, ew, emaskf, 16)
    rspec = pl.BlockSpec((8, RB, 128), lambda i: (1, i, 0))
    h = _post12(agg, hm, rspec, b1, g1, be1, r_slabbed=True)

    Wc = _wcat(W2, as2, ad2)
    bias = jnp.concatenate([zero8, zero128])
    hm = _mm(h, Wc, bias)
    agg = _edge_phase(hm, esrc, edst, ew, emaskf, 8)
    rspec = pl.BlockSpec((RB, HC), lambda i: (i, 0))
    h = _post12(agg, h, rspec, b2, g2, be2, r_slabbed=False)

    Wc = _wcat(W3, as3, ad3, Wr2)
    bias = jnp.concatenate([zero8, br2, zero128])
    hm = _mm(h, Wc, bias)
    agg = _edge_phase(hm, esrc, edst, ew, emaskf, 10)
    rspec = pl.BlockSpec((2, RB, 128), lambda i: (4, i, 0))
    wf_pad = jnp.pad(Wf, ((0, 0), (0, 128 - NUM_CLASSES)))
    bf_pad = jnp.pad(bf, (0, 128 - NUM_CLASSES))[None]
    h = _post34(agg, hm, rspec, b3, g3, be3, True, False, wf_pad, bf_pad)

    Wc = _wcat(W4, as4, ad4)
    bias = jnp.concatenate([zero8, zero128])
    hm = _mm(h, Wc, bias)
    agg = _edge_phase(hm, esrc, edst, ew, emaskf, 8)
    rspec = pl.BlockSpec((RB, HIDDEN), lambda i: (i, 0))
    out = _post34(agg, h, rspec, b4, g4, be4, False, True, wf_pad, bf_pad)

    return out[:N, :NUM_CLASSES]
